# Initial kernel scaffold; baseline (speedup 1.0000x reference)
import jax
import jax.numpy as jnp
from jax.experimental import pallas as pl


def kernel(pos, edge_index, node_type, edge_type, W_fourier, node_table, edge_table, in_W, in_b, dist_W, dist_b, proj_W1, proj_b1, proj_W2, proj_b2):
    raise NotImplementedError("write your pallas kernel here")


# speedup vs baseline: 1.5413x; 1.5413x over previous
import functools

import jax
import jax.numpy as jnp
from jax import lax
from jax.experimental import pallas as pl
from jax.experimental.pallas import tpu as pltpu
from jax.experimental.pallas import tpu_sc as plsc

N_NODES = 10000
N_EDGES = 160000
E_PAD = 163840
HID = 256
NC, NS = 2, 16
PER_W = E_PAD // (NC * NS)
CK = 1024
EB = 512
PER_T = E_PAD // NS
ROWS_PER_T = 625

_mesh = lambda: plsc.VectorSubcoreMesh(
    core_axis_name="c", subcore_axis_name="s", num_cores=NC, num_subcores=NS)


def _gather_body(posx, posy, posz, ntf, rowp, colp, g_out,
                 ridx, cidx, gbuf, sem):
  wid = lax.axis_index("s") * NC + lax.axis_index("c")
  for ch in range(PER_W // CK):
    base = wid * PER_W + ch * CK
    pltpu.sync_copy(rowp.at[pl.ds(base, CK)], ridx)
    pltpu.sync_copy(colp.at[pl.ds(base, CK)], cidx)
    descs = []
    for k in range(CK // 128):
      isl = pl.ds(k * 128, 128)
      for c, tab in enumerate((posx, posy, posz)):
        descs.append(
            pltpu.async_copy(tab.at[ridx.at[isl]], gbuf.at[c, isl], sem))
        descs.append(
            pltpu.async_copy(tab.at[cidx.at[isl]], gbuf.at[3 + c, isl], sem))
      descs.append(
          pltpu.async_copy(ntf.at[ridx.at[isl]], gbuf.at[6, isl], sem))
      descs.append(
          pltpu.async_copy(ntf.at[cidx.at[isl]], gbuf.at[7, isl], sem))
    for dsc in descs:
      dsc.wait()
    pltpu.sync_copy(gbuf, g_out.at[:, pl.ds(base, CK)])


def _sc_gather(posx, posy, posz, ntf, rowp, colp):
  k = functools.partial(
      pl.kernel,
      out_type=jax.ShapeDtypeStruct((8, E_PAD), jnp.float32),
      mesh=_mesh(),
      scratch_types=[
          pltpu.VMEM((CK,), jnp.int32),
          pltpu.VMEM((CK,), jnp.int32),
          pltpu.VMEM((8, CK), jnp.float32),
          pltpu.SemaphoreType.DMA,
      ],
      compiler_params=pltpu.CompilerParams(needs_layout_passes=False),
  )(_gather_body)
  return k(posx, posy, posz, ntf, rowp, colp)


def _prep_body(nt_ref, w_ref, o_ref):
  o_ref[...] = jnp.dot(nt_ref[...], w_ref[...],
                       preferred_element_type=jnp.float32,
                       precision=jax.lax.Precision.HIGHEST)


def _tc_prep(ntp, w_cat):
  return pl.pallas_call(
      _prep_body,
      out_shape=jax.ShapeDtypeStruct((128, 2 * HID), jnp.float32),
  )(ntp, w_cat)


def _main_body(g_ref, et_ref, w2pi_ref, w1s_ref, w1c_ref, w1ab_ref, b1_ref,
               w2_ref, b2_ref, ab_ref, inb_ref, etab_ref, dw_ref, db_ref,
               msgl_ref, msgr_ref, eqx_ref, eqy_ref, eqz_ref):
  g = g_ref[...]
  prx, pry, prz = g[0:1, :], g[1:2, :], g[2:3, :]
  pcx, pcy, pcz = g[3:4, :], g[4:5, :], g[5:6, :]
  trow, tcol = g[6:7, :], g[7:8, :]
  et = et_ref[...]

  dx, dy, dz = prx - pcx, pry - pcy, prz - pcz
  radial = dx * dx + dy * dy + dz * dz
  d = jnp.sqrt(radial + 1e-4)
  norm = jnp.sqrt(radial + 1e-12) + 1.0
  dnx, dny, dnz = dx / norm, dy / norm, dz / norm
  cx = pry * pcz - prz * pcy
  cy = prz * pcx - prx * pcz
  cz = prx * pcy - pry * pcx
  cn = jnp.sqrt(cx * cx + cy * cy + cz * cz + 1e-12) + 1.0
  cnx, cny, cnz = cx / cn, cy / cn, cz / cn
  vx = dny * cnz - dnz * cny
  vy = dnz * cnx - dnx * cnz
  vz = dnx * cny - dny * cnx
  nr = jnp.sqrt(prx * prx + pry * pry + prz * prz) + 1e-5
  ncn = jnp.sqrt(pcx * pcx + pcy * pcy + pcz * pcz) + 1e-5
  cos_t = (prx * pcx + pry * pcy + prz * pcz) / (nr * ncn)
  sin_t = jnp.sqrt(jnp.clip(1.0 - cos_t * cos_t, 0.0, 1.0))

  eqx_ref[...] = (dnx + vx).reshape((EB,))
  eqy_ref[...] = (dny + vy).reshape((EB,))
  eqz_ref[...] = (dnz + vz).reshape((EB,))

  scal = jnp.concatenate(
      [d, cos_t, sin_t, trow, tcol, et, jnp.zeros((2, EB), jnp.float32)],
      axis=0)
  st = scal.T
  d_c = st[:, 0:1]
  angles = st[:, 1:3]
  trow_c, tcol_c, et_c = st[:, 3:4], st[:, 4:5], st[:, 5:6]

  f32 = jnp.float32
  dot = functools.partial(jax.lax.dot_general, preferred_element_type=f32,
                          precision=jax.lax.Precision.HIGHEST)
  dotx = functools.partial(jax.lax.dot_general, preferred_element_type=f32,
                           precision=jax.lax.Precision.HIGHEST)
  nn = (((1,), (0,)), ((), ()))

  x = dotx(d_c, w2pi_ref[...], nn)
  p1 = (dot(jnp.sin(x), w1s_ref[...], nn)
        + dot(jnp.cos(x), w1c_ref[...], nn)
        + dotx(angles, w1ab_ref[...], nn)
        + b1_ref[...])
  p = dot(jnp.maximum(p1, 0.0), w2_ref[...], nn) + b2_ref[...]

  lane = lax.broadcasted_iota(jnp.int32, (EB, 128), 1).astype(f32)
  ohr = (lane == trow_c).astype(f32)
  ohc = (lane == tcol_c).astype(f32)
  ohe = (lane == et_c).astype(f32)
  ab = ab_ref[...]
  he = jnp.maximum(
      dot(ohr, ab[:, :HID], nn) + dot(ohc, ab[:, HID:], nn) + inb_ref[...],
      0.0)
  ea = dot(ohe, etab_ref[...], nn)
  dml = jnp.maximum(dotx(d_c, dw_ref[...], nn) + db_ref[...], 0.0)

  msg = he * ea * dml + p
  msgl_ref[...] = msg[:, :128]
  msgr_ref[...] = msg[:, 128:]


def _tc_main(g_arr, et_row, w2pi, w1s, w1c, w1ab, b1, w2, b2, ab, inb, etab,
             dw, db):
  nblk = E_PAD // EB
  full = lambda shape: pl.BlockSpec(shape, lambda i: (0,) * len(shape))
  return pl.pallas_call(
      _main_body,
      grid=(nblk,),
      in_specs=[
          pl.BlockSpec((8, EB), lambda i: (0, i)),
          pl.BlockSpec((1, EB), lambda i: (0, i)),
          full((1, HID)), full((HID, HID)), full((HID, HID)),
          full((2, HID)), full((1, HID)), full((HID, HID)), full((1, HID)),
          full((128, 2 * HID)), full((1, HID)), full((128, HID)),
          full((1, HID)), full((1, HID)),
      ],
      out_specs=[
          pl.BlockSpec((EB, 128), lambda i: (i, 0)),
          pl.BlockSpec((EB, 128), lambda i: (i, 0)),
          pl.BlockSpec((EB,), lambda i: (i,)),
          pl.BlockSpec((EB,), lambda i: (i,)),
          pl.BlockSpec((EB,), lambda i: (i,)),
      ],
      out_shape=[
          jax.ShapeDtypeStruct((E_PAD, 128), jnp.float32),
          jax.ShapeDtypeStruct((E_PAD, 128), jnp.float32),
          jax.ShapeDtypeStruct((E_PAD,), jnp.float32),
          jax.ShapeDtypeStruct((E_PAD,), jnp.float32),
          jax.ShapeDtypeStruct((E_PAD,), jnp.float32),
      ],
      compiler_params=pltpu.CompilerParams(
          dimension_semantics=("arbitrary",)),
  )(g_arr, et_row, w2pi, w1s, w1c, w1ab, b1, w2, b2, ab, inb, etab, dw, db)


HALF_N = N_NODES // 2
ACC_ROWS = 6144
W_ROWS = 312


def _scatter_body(zeros_hbm, msgl, msgr, eqxa, eqya, eqza, colp,
                  featl, featr, eqout, speq,
                  idxv, idxw, rows, eqb0, eqb1, eqb2, eqa0, eqa1, eqa2, zsrc,
                  eqred, acc):
  cid = lax.axis_index("c")
  t = lax.axis_index("s")

  pltpu.sync_copy(zeros_hbm, zsrc)

  def zeq(i, carry):
    z = jnp.zeros((16,), jnp.float32)
    eqa0[pl.ds(i * 16, 16)] = z
    eqa1[pl.ds(i * 16, 16)] = z
    eqa2[pl.ds(i * 16, 16)] = z
    return carry
  lax.fori_loop(0, N_NODES // 16, zeq, 0)

  start = t * PER_T
  n_ch = jnp.minimum(N_EDGES - start, PER_T) // 128

  def run(msgx, h, with_eq):
    def chunk(j, carry):
      base = start + j * 128
      pltpu.sync_copy(colp.at[pl.ds(base, 128)], idxv)
      pltpu.sync_copy(msgx.at[pl.ds(base, 128), :], rows)
      if with_eq:
        pltpu.sync_copy(eqxa.at[pl.ds(base, 128)], eqb0)
        pltpu.sync_copy(eqya.at[pl.ds(base, 128)], eqb1)
        pltpu.sync_copy(eqza.at[pl.ds(base, 128)], eqb2)
      for k in range(8):
        sl = pl.ds(k * 16, 16)
        c16 = idxv[sl]
        local = c16 - h * HALF_N
        valid = jnp.logical_and(local >= 0, local < HALF_N)
        idxw[sl] = jnp.where(valid, local,
                             HALF_N + jnp.bitwise_and(c16, 1023))
        if with_eq:
          plsc.addupdate_scatter(eqa0, [c16], eqb0[sl])
          plsc.addupdate_scatter(eqa1, [c16], eqb1[sl])
          plsc.addupdate_scatter(eqa2, [c16], eqb2[sl])
      pltpu.sync_copy(rows, acc.at[idxw], add=True)
      return carry
    lax.fori_loop(0, n_ch, chunk, 0)

  for h in range(2):
    for j in range(3):
      pltpu.sync_copy(zsrc, acc.at[pl.ds(t * 384 + j * 128, 128)])
    plsc.subcore_barrier()

    @pl.when(cid == 0)
    def _():
      run(msgl, h, with_eq=(h == 0))
      if h == 0:
        pltpu.sync_copy(eqa0, speq.at[pl.ds(t * 3 * N_NODES, N_NODES)])
        pltpu.sync_copy(eqa1,
                        speq.at[pl.ds(t * 3 * N_NODES + N_NODES, N_NODES)])
        pltpu.sync_copy(eqa2,
                        speq.at[pl.ds(t * 3 * N_NODES + 2 * N_NODES,
                                      N_NODES)])

    @pl.when(cid == 1)
    def _():
      run(msgr, h, with_eq=False)

    plsc.subcore_barrier()

    def writeout(featx, rstart, nrows):
      pltpu.sync_copy(acc.at[pl.ds(rstart, nrows)],
                      featx.at[pl.ds(h * HALF_N + rstart, nrows)])

    @pl.when(jnp.logical_and(cid == 0, t < NS - 1))
    def _():
      writeout(featl, t * W_ROWS, W_ROWS)
    @pl.when(jnp.logical_and(cid == 0, t == NS - 1))
    def _():
      writeout(featl, (NS - 1) * W_ROWS, HALF_N - (NS - 1) * W_ROWS)
    @pl.when(jnp.logical_and(cid == 1, t < NS - 1))
    def _():
      writeout(featr, t * W_ROWS, W_ROWS)
    @pl.when(jnp.logical_and(cid == 1, t == NS - 1))
    def _():
      writeout(featr, (NS - 1) * W_ROWS, HALF_N - (NS - 1) * W_ROWS)
    plsc.subcore_barrier()

  def eq_reduce(nbase, cnt):
    for r, eqa in ((0, eqa0), (1, eqa1), (2, eqa2)):
      pltpu.sync_copy(speq.at[pl.ds(r * N_NODES + nbase, cnt)],
                      eqa.at[pl.ds(0, cnt)])
      for src in range(1, NS):
        pltpu.sync_copy(
            speq.at[pl.ds((src * 3 + r) * N_NODES + nbase, cnt)],
            eqred.at[pl.ds(0, cnt)])

        def addv(i, carry):
          sl16 = pl.ds(i * 16, 16)
          eqa[sl16] = eqa[sl16] + eqred[sl16]
          return carry
        lax.fori_loop(0, cnt // 16, addv, 0)
      pltpu.sync_copy(eqa.at[pl.ds(0, cnt)],
                      eqout.at[pl.ds(r * N_NODES + nbase, cnt)])

  @pl.when(jnp.logical_and(cid == 0, t < NS - 1))
  def _():
    eq_reduce(t * 640, 640)

  @pl.when(jnp.logical_and(cid == 0, t == NS - 1))
  def _():
    eq_reduce((NS - 1) * 640, 400)


def _sc_scatter(zeros, msgl, msgr, eqxa, eqya, eqza, colp):
  k = functools.partial(
      pl.kernel,
      out_type=(
          jax.ShapeDtypeStruct((N_NODES, 128), jnp.float32),
          jax.ShapeDtypeStruct((N_NODES, 128), jnp.float32),
          jax.ShapeDtypeStruct((3 * N_NODES,), jnp.float32),
          jax.ShapeDtypeStruct((NS * 3 * N_NODES,), jnp.float32),
      ),
      mesh=_mesh(),
      scratch_types=[
          pltpu.VMEM((128,), jnp.int32),
          pltpu.VMEM((128,), jnp.int32),
          pltpu.VMEM((128, 128), jnp.float32),
          pltpu.VMEM((128,), jnp.float32),
          pltpu.VMEM((128,), jnp.float32),
          pltpu.VMEM((128,), jnp.float32),
          pltpu.VMEM((N_NODES,), jnp.float32),
          pltpu.VMEM((N_NODES,), jnp.float32),
          pltpu.VMEM((N_NODES,), jnp.float32),
          pltpu.VMEM((128, 128), jnp.float32),
          pltpu.VMEM((640,), jnp.float32),
          pltpu.VMEM_SHARED((ACC_ROWS, 128), jnp.float32),
      ],
      compiler_params=pltpu.CompilerParams(needs_layout_passes=False),
  )(_scatter_body)
  return k(zeros, msgl, msgr, eqxa, eqya, eqza, colp)


def kernel(pos, edge_index, node_type, edge_type, W_fourier, node_table,
           edge_table, in_W, in_b, dist_W, dist_b, proj_W1, proj_b1, proj_W2,
           proj_b2):
  f32 = jnp.float32
  pad_e = E_PAD - N_EDGES
  row = edge_index[0]
  col = edge_index[1]
  rowp = jnp.concatenate([row, jnp.zeros((pad_e,), row.dtype)])
  colp = jnp.concatenate([col, jnp.zeros((pad_e,), col.dtype)])
  posx, posy, posz = pos[:, 0], pos[:, 1], pos[:, 2]
  ntf = node_type.astype(f32)
  et_row = jnp.pad(edge_type.astype(f32), (0, pad_e)).reshape(1, E_PAD)

  ntp = jnp.concatenate([node_table, jnp.zeros((28, HID), f32)], axis=0)
  etab = jnp.concatenate([edge_table, jnp.zeros((28, HID), f32)], axis=0)
  w_cat = jnp.concatenate([in_W[:HID], in_W[HID:]], axis=1)
  w2pi = (W_fourier * (2.0 * jnp.pi)).reshape(1, HID)
  w1s, w1c = proj_W1[:HID], proj_W1[HID:2 * HID]
  w1ab = proj_W1[2 * HID:]
  b1 = proj_b1.reshape(1, HID)
  b2 = proj_b2.reshape(1, HID)
  inb = in_b.reshape(1, HID)
  db = dist_b.reshape(1, HID)

  ab = _tc_prep(ntp, w_cat)
  g_arr = _sc_gather(posx, posy, posz, ntf, rowp, colp)
  msgl, msgr, eqxa, eqya, eqza = _tc_main(g_arr, et_row, w2pi, w1s, w1c,
                                          w1ab, b1, proj_W2, b2, ab, inb,
                                          etab, dist_W, db)
  featl, featr, eqout, _ = _sc_scatter(jnp.zeros((128, 128), f32), msgl,
                                       msgr, eqxa, eqya, eqza, colp)
  return jnp.concatenate([featl, featr, eqout.reshape(3, N_NODES).T], axis=1)


# speedup vs baseline: 2.3030x; 1.4942x over previous
import functools

import jax
import jax.numpy as jnp
from jax import lax
from jax.experimental import pallas as pl
from jax.experimental.pallas import tpu as pltpu
from jax.experimental.pallas import tpu_sc as plsc

N_NODES = 10000
N_EDGES = 160000
E_PAD = 163840
HID = 256
NC, NS = 2, 16
PER_W = E_PAD // (NC * NS)
CK = 1024
EB = 512
PER_T = E_PAD // NS
ROWS_PER_T = 625

_mesh = lambda: plsc.VectorSubcoreMesh(
    core_axis_name="c", subcore_axis_name="s", num_cores=NC, num_subcores=NS)


def _gather_body(posx, posy, posz, ntf, rowp, colp, g_out,
                 ridx, cidx, gbuf, sem):
  wid = lax.axis_index("s") * NC + lax.axis_index("c")
  for ch in range(PER_W // CK):
    base = wid * PER_W + ch * CK
    pltpu.sync_copy(rowp.at[pl.ds(base, CK)], ridx)
    pltpu.sync_copy(colp.at[pl.ds(base, CK)], cidx)
    descs = []
    for k in range(CK // 128):
      isl = pl.ds(k * 128, 128)
      for c, tab in enumerate((posx, posy, posz)):
        descs.append(
            pltpu.async_copy(tab.at[ridx.at[isl]], gbuf.at[c, isl], sem))
        descs.append(
            pltpu.async_copy(tab.at[cidx.at[isl]], gbuf.at[3 + c, isl], sem))
      descs.append(
          pltpu.async_copy(ntf.at[ridx.at[isl]], gbuf.at[6, isl], sem))
      descs.append(
          pltpu.async_copy(ntf.at[cidx.at[isl]], gbuf.at[7, isl], sem))
    for dsc in descs:
      dsc.wait()
    pltpu.sync_copy(gbuf, g_out.at[:, pl.ds(base, CK)])


def _sc_gather(posx, posy, posz, ntf, rowp, colp):
  k = functools.partial(
      pl.kernel,
      out_type=jax.ShapeDtypeStruct((8, E_PAD), jnp.float32),
      mesh=_mesh(),
      scratch_types=[
          pltpu.VMEM((CK,), jnp.int32),
          pltpu.VMEM((CK,), jnp.int32),
          pltpu.VMEM((8, CK), jnp.float32),
          pltpu.SemaphoreType.DMA,
      ],
      compiler_params=pltpu.CompilerParams(needs_layout_passes=False),
  )(_gather_body)
  return k(posx, posy, posz, ntf, rowp, colp)


def _prep_body(nt_ref, w_ref, o_ref):
  o_ref[...] = jnp.dot(nt_ref[...], w_ref[...],
                       preferred_element_type=jnp.float32,
                       precision=jax.lax.Precision.HIGHEST)


def _tc_prep(ntp, w_cat):
  return pl.pallas_call(
      _prep_body,
      out_shape=jax.ShapeDtypeStruct((128, 2 * HID), jnp.float32),
  )(ntp, w_cat)


def _main_body(g_ref, et_ref, w2pi_ref, w1s_ref, w1c_ref, w1ab_ref, b1_ref,
               w2_ref, b2_ref, ab_ref, inb_ref, etab_ref, dw_ref, db_ref,
               msgl_ref, msgr_ref, eqx_ref, eqy_ref, eqz_ref):
  g = g_ref[...]
  prx, pry, prz = g[0:1, :], g[1:2, :], g[2:3, :]
  pcx, pcy, pcz = g[3:4, :], g[4:5, :], g[5:6, :]
  trow, tcol = g[6:7, :], g[7:8, :]
  et = et_ref[...]

  dx, dy, dz = prx - pcx, pry - pcy, prz - pcz
  radial = dx * dx + dy * dy + dz * dz
  d = jnp.sqrt(radial + 1e-4)
  norm = jnp.sqrt(radial + 1e-12) + 1.0
  dnx, dny, dnz = dx / norm, dy / norm, dz / norm
  cx = pry * pcz - prz * pcy
  cy = prz * pcx - prx * pcz
  cz = prx * pcy - pry * pcx
  cn = jnp.sqrt(cx * cx + cy * cy + cz * cz + 1e-12) + 1.0
  cnx, cny, cnz = cx / cn, cy / cn, cz / cn
  vx = dny * cnz - dnz * cny
  vy = dnz * cnx - dnx * cnz
  vz = dnx * cny - dny * cnx
  nr = jnp.sqrt(prx * prx + pry * pry + prz * prz) + 1e-5
  ncn = jnp.sqrt(pcx * pcx + pcy * pcy + pcz * pcz) + 1e-5
  cos_t = (prx * pcx + pry * pcy + prz * pcz) / (nr * ncn)
  sin_t = jnp.sqrt(jnp.clip(1.0 - cos_t * cos_t, 0.0, 1.0))

  eqx_ref[...] = (dnx + vx).reshape((EB,))
  eqy_ref[...] = (dny + vy).reshape((EB,))
  eqz_ref[...] = (dnz + vz).reshape((EB,))

  scal = jnp.concatenate(
      [d, cos_t, sin_t, trow, tcol, et, jnp.zeros((2, EB), jnp.float32)],
      axis=0)
  st = scal.T
  d_c = st[:, 0:1]
  angles = st[:, 1:3]
  trow_c, tcol_c, et_c = st[:, 3:4], st[:, 4:5], st[:, 5:6]

  f32 = jnp.float32
  dot = functools.partial(jax.lax.dot_general, preferred_element_type=f32,
                          precision=jax.lax.Precision.DEFAULT)
  dotx = functools.partial(jax.lax.dot_general, preferred_element_type=f32,
                           precision=jax.lax.Precision.HIGHEST)
  nn = (((1,), (0,)), ((), ()))

  x = dotx(d_c, w2pi_ref[...], nn)
  p1 = (dot(jnp.sin(x), w1s_ref[...], nn)
        + dot(jnp.cos(x), w1c_ref[...], nn)
        + dotx(angles, w1ab_ref[...], nn)
        + b1_ref[...])
  p = dot(jnp.maximum(p1, 0.0), w2_ref[...], nn) + b2_ref[...]

  lane = lax.broadcasted_iota(jnp.int32, (EB, 128), 1).astype(f32)
  ohr = (lane == trow_c).astype(f32)
  ohc = (lane == tcol_c).astype(f32)
  ohe = (lane == et_c).astype(f32)
  ab = ab_ref[...]
  he = jnp.maximum(
      dot(ohr, ab[:, :HID], nn) + dot(ohc, ab[:, HID:], nn) + inb_ref[...],
      0.0)
  ea = dot(ohe, etab_ref[...], nn)
  dml = jnp.maximum(dotx(d_c, dw_ref[...], nn) + db_ref[...], 0.0)

  msg = he * ea * dml + p
  msgl_ref[...] = msg[:, :128]
  msgr_ref[...] = msg[:, 128:]


def _tc_main(g_arr, et_row, w2pi, w1s, w1c, w1ab, b1, w2, b2, ab, inb, etab,
             dw, db):
  nblk = E_PAD // EB
  full = lambda shape: pl.BlockSpec(shape, lambda i: (0,) * len(shape))
  return pl.pallas_call(
      _main_body,
      grid=(nblk,),
      in_specs=[
          pl.BlockSpec((8, EB), lambda i: (0, i)),
          pl.BlockSpec((1, EB), lambda i: (0, i)),
          full((1, HID)), full((HID, HID)), full((HID, HID)),
          full((2, HID)), full((1, HID)), full((HID, HID)), full((1, HID)),
          full((128, 2 * HID)), full((1, HID)), full((128, HID)),
          full((1, HID)), full((1, HID)),
      ],
      out_specs=[
          pl.BlockSpec((EB, 128), lambda i: (i, 0)),
          pl.BlockSpec((EB, 128), lambda i: (i, 0)),
          pl.BlockSpec((EB,), lambda i: (i,)),
          pl.BlockSpec((EB,), lambda i: (i,)),
          pl.BlockSpec((EB,), lambda i: (i,)),
      ],
      out_shape=[
          jax.ShapeDtypeStruct((E_PAD, 128), jnp.float32),
          jax.ShapeDtypeStruct((E_PAD, 128), jnp.float32),
          jax.ShapeDtypeStruct((E_PAD,), jnp.float32),
          jax.ShapeDtypeStruct((E_PAD,), jnp.float32),
          jax.ShapeDtypeStruct((E_PAD,), jnp.float32),
      ],
      compiler_params=pltpu.CompilerParams(
          dimension_semantics=("arbitrary",)),
  )(g_arr, et_row, w2pi, w1s, w1c, w1ab, b1, w2, b2, ab, inb, etab, dw, db)


HALF_N = N_NODES // 2
ACC_ROWS = 6144
W_ROWS = 312


def _scatter_body(zeros_hbm, msgl, msgr, eqxa, eqya, eqza, colp,
                  featl, featr, eqout, speq,
                  idxv, idxw, rows, eqb0, eqb1, eqb2, eqa0, eqa1, eqa2, zsrc,
                  eqred, acc):
  cid = lax.axis_index("c")
  t = lax.axis_index("s")

  pltpu.sync_copy(zeros_hbm, zsrc)

  def zeq(i, carry):
    z = jnp.zeros((16,), jnp.float32)
    eqa0[pl.ds(i * 16, 16)] = z
    eqa1[pl.ds(i * 16, 16)] = z
    eqa2[pl.ds(i * 16, 16)] = z
    return carry
  lax.fori_loop(0, N_NODES // 16, zeq, 0)

  start = t * PER_T
  n_ch = jnp.minimum(N_EDGES - start, PER_T) // 128

  def run(msgx, h, with_eq):
    def chunk(j, carry):
      base = start + j * 128
      pltpu.sync_copy(colp.at[pl.ds(base, 128)], idxv)
      pltpu.sync_copy(msgx.at[pl.ds(base, 128), :], rows)
      if with_eq:
        pltpu.sync_copy(eqxa.at[pl.ds(base, 128)], eqb0)
        pltpu.sync_copy(eqya.at[pl.ds(base, 128)], eqb1)
        pltpu.sync_copy(eqza.at[pl.ds(base, 128)], eqb2)
      for k in range(8):
        sl = pl.ds(k * 16, 16)
        c16 = idxv[sl]
        local = c16 - h * HALF_N
        valid = jnp.logical_and(local >= 0, local < HALF_N)
        idxw[sl] = jnp.where(valid, local,
                             HALF_N + jnp.bitwise_and(c16, 1023))
        if with_eq:
          plsc.addupdate_scatter(eqa0, [c16], eqb0[sl])
          plsc.addupdate_scatter(eqa1, [c16], eqb1[sl])
          plsc.addupdate_scatter(eqa2, [c16], eqb2[sl])
      pltpu.sync_copy(rows, acc.at[idxw], add=True)
      return carry
    lax.fori_loop(0, n_ch, chunk, 0)

  for h in range(2):
    for j in range(3):
      pltpu.sync_copy(zsrc, acc.at[pl.ds(t * 384 + j * 128, 128)])
    plsc.subcore_barrier()

    @pl.when(cid == 0)
    def _():
      run(msgl, h, with_eq=(h == 0))
      if h == 0:
        pltpu.sync_copy(eqa0, speq.at[pl.ds(t * 3 * N_NODES, N_NODES)])
        pltpu.sync_copy(eqa1,
                        speq.at[pl.ds(t * 3 * N_NODES + N_NODES, N_NODES)])
        pltpu.sync_copy(eqa2,
                        speq.at[pl.ds(t * 3 * N_NODES + 2 * N_NODES,
                                      N_NODES)])

    @pl.when(cid == 1)
    def _():
      run(msgr, h, with_eq=False)

    plsc.subcore_barrier()

    def writeout(featx, rstart, nrows):
      pltpu.sync_copy(acc.at[pl.ds(rstart, nrows)],
                      featx.at[pl.ds(h * HALF_N + rstart, nrows)])

    @pl.when(jnp.logical_and(cid == 0, t < NS - 1))
    def _():
      writeout(featl, t * W_ROWS, W_ROWS)
    @pl.when(jnp.logical_and(cid == 0, t == NS - 1))
    def _():
      writeout(featl, (NS - 1) * W_ROWS, HALF_N - (NS - 1) * W_ROWS)
    @pl.when(jnp.logical_and(cid == 1, t < NS - 1))
    def _():
      writeout(featr, t * W_ROWS, W_ROWS)
    @pl.when(jnp.logical_and(cid == 1, t == NS - 1))
    def _():
      writeout(featr, (NS - 1) * W_ROWS, HALF_N - (NS - 1) * W_ROWS)
    plsc.subcore_barrier()

  def eq_reduce(nbase, cnt):
    for r, eqa in ((0, eqa0), (1, eqa1), (2, eqa2)):
      pltpu.sync_copy(speq.at[pl.ds(r * N_NODES + nbase, cnt)],
                      eqa.at[pl.ds(0, cnt)])
      for src in range(1, NS):
        pltpu.sync_copy(
            speq.at[pl.ds((src * 3 + r) * N_NODES + nbase, cnt)],
            eqred.at[pl.ds(0, cnt)])

        def addv(i, carry):
          sl16 = pl.ds(i * 16, 16)
          eqa[sl16] = eqa[sl16] + eqred[sl16]
          return carry
        lax.fori_loop(0, cnt // 16, addv, 0)
      pltpu.sync_copy(eqa.at[pl.ds(0, cnt)],
                      eqout.at[pl.ds(r * N_NODES + nbase, cnt)])

  @pl.when(jnp.logical_and(cid == 0, t < NS - 1))
  def _():
    eq_reduce(t * 640, 640)

  @pl.when(jnp.logical_and(cid == 0, t == NS - 1))
  def _():
    eq_reduce((NS - 1) * 640, 400)


def _sc_scatter(zeros, msgl, msgr, eqxa, eqya, eqza, colp):
  k = functools.partial(
      pl.kernel,
      out_type=(
          jax.ShapeDtypeStruct((N_NODES, 128), jnp.float32),
          jax.ShapeDtypeStruct((N_NODES, 128), jnp.float32),
          jax.ShapeDtypeStruct((3 * N_NODES,), jnp.float32),
          jax.ShapeDtypeStruct((NS * 3 * N_NODES,), jnp.float32),
      ),
      mesh=_mesh(),
      scratch_types=[
          pltpu.VMEM((128,), jnp.int32),
          pltpu.VMEM((128,), jnp.int32),
          pltpu.VMEM((128, 128), jnp.float32),
          pltpu.VMEM((128,), jnp.float32),
          pltpu.VMEM((128,), jnp.float32),
          pltpu.VMEM((128,), jnp.float32),
          pltpu.VMEM((N_NODES,), jnp.float32),
          pltpu.VMEM((N_NODES,), jnp.float32),
          pltpu.VMEM((N_NODES,), jnp.float32),
          pltpu.VMEM((128, 128), jnp.float32),
          pltpu.VMEM((640,), jnp.float32),
          pltpu.VMEM_SHARED((ACC_ROWS, 128), jnp.float32),
      ],
      compiler_params=pltpu.CompilerParams(needs_layout_passes=False),
  )(_scatter_body)
  return k(zeros, msgl, msgr, eqxa, eqya, eqza, colp)


def kernel(pos, edge_index, node_type, edge_type, W_fourier, node_table,
           edge_table, in_W, in_b, dist_W, dist_b, proj_W1, proj_b1, proj_W2,
           proj_b2):
  f32 = jnp.float32
  pad_e = E_PAD - N_EDGES
  row = edge_index[0]
  col = edge_index[1]
  rowp = jnp.concatenate([row, jnp.zeros((pad_e,), row.dtype)])
  colp = jnp.concatenate([col, jnp.zeros((pad_e,), col.dtype)])
  posx, posy, posz = pos[:, 0], pos[:, 1], pos[:, 2]
  ntf = node_type.astype(f32)
  et_row = jnp.pad(edge_type.astype(f32), (0, pad_e)).reshape(1, E_PAD)

  ntp = jnp.concatenate([node_table, jnp.zeros((28, HID), f32)], axis=0)
  etab = jnp.concatenate([edge_table, jnp.zeros((28, HID), f32)], axis=0)
  w_cat = jnp.concatenate([in_W[:HID], in_W[HID:]], axis=1)
  w2pi = (W_fourier * (2.0 * jnp.pi)).reshape(1, HID)
  w1s, w1c = proj_W1[:HID], proj_W1[HID:2 * HID]
  w1ab = proj_W1[2 * HID:]
  b1 = proj_b1.reshape(1, HID)
  b2 = proj_b2.reshape(1, HID)
  inb = in_b.reshape(1, HID)
  db = dist_b.reshape(1, HID)

  ab = _tc_prep(ntp, w_cat)
  g_arr = _sc_gather(posx, posy, posz, ntf, rowp, colp)
  msgl, msgr, eqxa, eqya, eqza = _tc_main(g_arr, et_row, w2pi, w1s, w1c,
                                          w1ab, b1, proj_W2, b2, ab, inb,
                                          etab, dist_W, db)
  featl, featr, eqout, _ = _sc_scatter(jnp.zeros((128, 128), f32), msgl,
                                       msgr, eqxa, eqya, eqza, colp)
  return jnp.concatenate([featl, featr, eqout.reshape(3, N_NODES).T], axis=1)


# speedup vs baseline: 2.8650x; 1.2440x over previous
import functools

import jax
import jax.numpy as jnp
from jax import lax
from jax.experimental import pallas as pl
from jax.experimental.pallas import tpu as pltpu
from jax.experimental.pallas import tpu_sc as plsc

N_NODES = 10000
N_EDGES = 160000
E_PAD = 163840
HID = 256
NC, NS = 2, 16
PER_W = E_PAD // (NC * NS)
CK = 1024
EB = 512
PER_T = E_PAD // NS
ROWS_PER_T = 625

_mesh = lambda: plsc.VectorSubcoreMesh(
    core_axis_name="c", subcore_axis_name="s", num_cores=NC, num_subcores=NS)


def _gather_body(posx, posy, posz, ntf, rowp, colp, g_out,
                 ridx, cidx, gbuf, sem):
  wid = lax.axis_index("s") * NC + lax.axis_index("c")
  for ch in range(PER_W // CK):
    base = wid * PER_W + ch * CK
    pltpu.sync_copy(rowp.at[pl.ds(base, CK)], ridx)
    pltpu.sync_copy(colp.at[pl.ds(base, CK)], cidx)
    descs = []
    for k in range(CK // 128):
      isl = pl.ds(k * 128, 128)
      for c, tab in enumerate((posx, posy, posz)):
        descs.append(
            pltpu.async_copy(tab.at[ridx.at[isl]], gbuf.at[c, isl], sem))
        descs.append(
            pltpu.async_copy(tab.at[cidx.at[isl]], gbuf.at[3 + c, isl], sem))
      descs.append(
          pltpu.async_copy(ntf.at[ridx.at[isl]], gbuf.at[6, isl], sem))
      descs.append(
          pltpu.async_copy(ntf.at[cidx.at[isl]], gbuf.at[7, isl], sem))
    for dsc in descs:
      dsc.wait()
    pltpu.sync_copy(gbuf, g_out.at[:, pl.ds(base, CK)])


def _sc_gather(posx, posy, posz, ntf, rowp, colp):
  k = functools.partial(
      pl.kernel,
      out_type=jax.ShapeDtypeStruct((8, E_PAD), jnp.float32),
      mesh=_mesh(),
      scratch_types=[
          pltpu.VMEM((CK,), jnp.int32),
          pltpu.VMEM((CK,), jnp.int32),
          pltpu.VMEM((8, CK), jnp.float32),
          pltpu.SemaphoreType.DMA,
      ],
      compiler_params=pltpu.CompilerParams(needs_layout_passes=False),
  )(_gather_body)
  return k(posx, posy, posz, ntf, rowp, colp)


def _prep_body(nt_ref, w_ref, o_ref):
  o_ref[...] = jnp.dot(nt_ref[...], w_ref[...],
                       preferred_element_type=jnp.float32,
                       precision=jax.lax.Precision.HIGHEST)


def _tc_prep(ntp, w_cat):
  return pl.pallas_call(
      _prep_body,
      out_shape=jax.ShapeDtypeStruct((128, 2 * HID), jnp.float32),
  )(ntp, w_cat)


def _main_body(g_ref, et_ref, w2pi_ref, w1s_ref, w1c_ref, w1ab_ref, b1_ref,
               w2_ref, b2_ref, ab_ref, inb_ref, etab_ref, dw_ref, db_ref,
               msgl_ref, msgr_ref, eqx_ref, eqy_ref, eqz_ref):
  g = g_ref[...]
  prx, pry, prz = g[0:1, :], g[1:2, :], g[2:3, :]
  pcx, pcy, pcz = g[3:4, :], g[4:5, :], g[5:6, :]
  trow, tcol = g[6:7, :], g[7:8, :]
  et = et_ref[...]

  dx, dy, dz = prx - pcx, pry - pcy, prz - pcz
  radial = dx * dx + dy * dy + dz * dz
  d = jnp.sqrt(radial + 1e-4)
  norm = jnp.sqrt(radial + 1e-12) + 1.0
  dnx, dny, dnz = dx / norm, dy / norm, dz / norm
  cx = pry * pcz - prz * pcy
  cy = prz * pcx - prx * pcz
  cz = prx * pcy - pry * pcx
  cn = jnp.sqrt(cx * cx + cy * cy + cz * cz + 1e-12) + 1.0
  cnx, cny, cnz = cx / cn, cy / cn, cz / cn
  vx = dny * cnz - dnz * cny
  vy = dnz * cnx - dnx * cnz
  vz = dnx * cny - dny * cnx
  nr = jnp.sqrt(prx * prx + pry * pry + prz * prz) + 1e-5
  ncn = jnp.sqrt(pcx * pcx + pcy * pcy + pcz * pcz) + 1e-5
  cos_t = (prx * pcx + pry * pcy + prz * pcz) / (nr * ncn)
  sin_t = jnp.sqrt(jnp.clip(1.0 - cos_t * cos_t, 0.0, 1.0))

  eqx_ref[...] = (dnx + vx).reshape((EB,))
  eqy_ref[...] = (dny + vy).reshape((EB,))
  eqz_ref[...] = (dnz + vz).reshape((EB,))

  scal = jnp.concatenate(
      [d, cos_t, sin_t, trow, tcol, et, jnp.zeros((2, EB), jnp.float32)],
      axis=0)
  st = scal.T
  d_c = st[:, 0:1]
  angles = st[:, 1:3]
  trow_c, tcol_c, et_c = st[:, 3:4], st[:, 4:5], st[:, 5:6]

  f32 = jnp.float32
  dot = functools.partial(jax.lax.dot_general, preferred_element_type=f32,
                          precision=jax.lax.Precision.DEFAULT)
  dotx = functools.partial(jax.lax.dot_general, preferred_element_type=f32,
                           precision=jax.lax.Precision.HIGHEST)
  nn = (((1,), (0,)), ((), ()))

  x = dotx(d_c, w2pi_ref[...], nn)
  n = jnp.round(x * 0.3183098861837907)
  r = x - n * 3.140625 - n * 9.67653589793e-4
  odd = jnp.bitwise_and(n.astype(jnp.int32), 1)
  sgn = jnp.where(odd == 1, -1.0, 1.0)
  r2 = r * r
  sp = r * (1.0 + r2 * (-1.6666654611e-1
                        + r2 * (8.3321608736e-3 + r2 * -1.9515295891e-4)))
  cp = (1.0 - 0.5 * r2
        + r2 * r2 * (4.166664568298827e-2
                     + r2 * (-1.388731625493765e-3
                             + r2 * 2.443315711809948e-5)))
  p1 = (dot(sgn * sp, w1s_ref[...], nn)
        + dot(sgn * cp, w1c_ref[...], nn)
        + dotx(angles, w1ab_ref[...], nn)
        + b1_ref[...])
  p = dot(jnp.maximum(p1, 0.0), w2_ref[...], nn) + b2_ref[...]

  lane = lax.broadcasted_iota(jnp.int32, (EB, 128), 1).astype(f32)
  ohr = (lane == trow_c).astype(f32)
  ohc = (lane == tcol_c).astype(f32)
  ohe = (lane == et_c).astype(f32)
  ab = ab_ref[...]
  he = jnp.maximum(
      dot(ohr, ab[:, :HID], nn) + dot(ohc, ab[:, HID:], nn) + inb_ref[...],
      0.0)
  ea = dot(ohe, etab_ref[...], nn)
  dml = jnp.maximum(dotx(d_c, dw_ref[...], nn) + db_ref[...], 0.0)

  msg = he * ea * dml + p
  msgl_ref[...] = msg[:, :128]
  msgr_ref[...] = msg[:, 128:]


def _tc_main(g_arr, et_row, w2pi, w1s, w1c, w1ab, b1, w2, b2, ab, inb, etab,
             dw, db):
  nblk = E_PAD // EB
  full = lambda shape: pl.BlockSpec(shape, lambda i: (0,) * len(shape))
  return pl.pallas_call(
      _main_body,
      grid=(nblk,),
      in_specs=[
          pl.BlockSpec((8, EB), lambda i: (0, i)),
          pl.BlockSpec((1, EB), lambda i: (0, i)),
          full((1, HID)), full((HID, HID)), full((HID, HID)),
          full((2, HID)), full((1, HID)), full((HID, HID)), full((1, HID)),
          full((128, 2 * HID)), full((1, HID)), full((128, HID)),
          full((1, HID)), full((1, HID)),
      ],
      out_specs=[
          pl.BlockSpec((EB, 128), lambda i: (i, 0)),
          pl.BlockSpec((EB, 128), lambda i: (i, 0)),
          pl.BlockSpec((EB,), lambda i: (i,)),
          pl.BlockSpec((EB,), lambda i: (i,)),
          pl.BlockSpec((EB,), lambda i: (i,)),
      ],
      out_shape=[
          jax.ShapeDtypeStruct((E_PAD, 128), jnp.float32),
          jax.ShapeDtypeStruct((E_PAD, 128), jnp.float32),
          jax.ShapeDtypeStruct((E_PAD,), jnp.float32),
          jax.ShapeDtypeStruct((E_PAD,), jnp.float32),
          jax.ShapeDtypeStruct((E_PAD,), jnp.float32),
      ],
      compiler_params=pltpu.CompilerParams(
          dimension_semantics=("arbitrary",)),
  )(g_arr, et_row, w2pi, w1s, w1c, w1ab, b1, w2, b2, ab, inb, etab, dw, db)


HALF_N = N_NODES // 2
ACC_ROWS = 6144
W_ROWS = 312


def _scatter_body(zeros_hbm, msgl, msgr, eqxa, eqya, eqza, colp,
                  featl, featr, eqout, speq,
                  idxv, idxw, rows, eqb0, eqb1, eqb2, eqa0, eqa1, eqa2, zsrc,
                  eqred, acc):
  cid = lax.axis_index("c")
  t = lax.axis_index("s")

  pltpu.sync_copy(zeros_hbm, zsrc)

  def zeq(i, carry):
    z = jnp.zeros((16,), jnp.float32)
    eqa0[pl.ds(i * 16, 16)] = z
    eqa1[pl.ds(i * 16, 16)] = z
    eqa2[pl.ds(i * 16, 16)] = z
    return carry
  lax.fori_loop(0, N_NODES // 16, zeq, 0)

  start = t * PER_T
  n_ch = jnp.minimum(N_EDGES - start, PER_T) // 128

  def run(msgx, h, with_eq):
    def chunk(j, carry):
      base = start + j * 128
      pltpu.sync_copy(colp.at[pl.ds(base, 128)], idxv)
      pltpu.sync_copy(msgx.at[pl.ds(base, 128), :], rows)
      if with_eq:
        pltpu.sync_copy(eqxa.at[pl.ds(base, 128)], eqb0)
        pltpu.sync_copy(eqya.at[pl.ds(base, 128)], eqb1)
        pltpu.sync_copy(eqza.at[pl.ds(base, 128)], eqb2)
      for k in range(8):
        sl = pl.ds(k * 16, 16)
        c16 = idxv[sl]
        local = c16 - h * HALF_N
        valid = jnp.logical_and(local >= 0, local < HALF_N)
        idxw[sl] = jnp.where(valid, local,
                             HALF_N + jnp.bitwise_and(c16, 1023))
        if with_eq:
          plsc.addupdate_scatter(eqa0, [c16], eqb0[sl])
          plsc.addupdate_scatter(eqa1, [c16], eqb1[sl])
          plsc.addupdate_scatter(eqa2, [c16], eqb2[sl])
      pltpu.sync_copy(rows, acc.at[idxw], add=True)
      return carry
    lax.fori_loop(0, n_ch, chunk, 0)

  for h in range(2):
    for j in range(3):
      pltpu.sync_copy(zsrc, acc.at[pl.ds(t * 384 + j * 128, 128)])
    plsc.subcore_barrier()

    @pl.when(cid == 0)
    def _():
      run(msgl, h, with_eq=(h == 0))
      if h == 0:
        pltpu.sync_copy(eqa0, speq.at[pl.ds(t * 3 * N_NODES, N_NODES)])
        pltpu.sync_copy(eqa1,
                        speq.at[pl.ds(t * 3 * N_NODES + N_NODES, N_NODES)])
        pltpu.sync_copy(eqa2,
                        speq.at[pl.ds(t * 3 * N_NODES + 2 * N_NODES,
                                      N_NODES)])

    @pl.when(cid == 1)
    def _():
      run(msgr, h, with_eq=False)

    plsc.subcore_barrier()

    def writeout(featx, rstart, nrows):
      pltpu.sync_copy(acc.at[pl.ds(rstart, nrows)],
                      featx.at[pl.ds(h * HALF_N + rstart, nrows)])

    @pl.when(jnp.logical_and(cid == 0, t < NS - 1))
    def _():
      writeout(featl, t * W_ROWS, W_ROWS)
    @pl.when(jnp.logical_and(cid == 0, t == NS - 1))
    def _():
      writeout(featl, (NS - 1) * W_ROWS, HALF_N - (NS - 1) * W_ROWS)
    @pl.when(jnp.logical_and(cid == 1, t < NS - 1))
    def _():
      writeout(featr, t * W_ROWS, W_ROWS)
    @pl.when(jnp.logical_and(cid == 1, t == NS - 1))
    def _():
      writeout(featr, (NS - 1) * W_ROWS, HALF_N - (NS - 1) * W_ROWS)
    plsc.subcore_barrier()

  def eq_reduce(nbase, cnt):
    for r, eqa in ((0, eqa0), (1, eqa1), (2, eqa2)):
      pltpu.sync_copy(speq.at[pl.ds(r * N_NODES + nbase, cnt)],
                      eqa.at[pl.ds(0, cnt)])
      for src in range(1, NS):
        pltpu.sync_copy(
            speq.at[pl.ds((src * 3 + r) * N_NODES + nbase, cnt)],
            eqred.at[pl.ds(0, cnt)])

        def addv(i, carry):
          sl16 = pl.ds(i * 16, 16)
          eqa[sl16] = eqa[sl16] + eqred[sl16]
          return carry
        lax.fori_loop(0, cnt // 16, addv, 0)
      pltpu.sync_copy(eqa.at[pl.ds(0, cnt)],
                      eqout.at[pl.ds(r * N_NODES + nbase, cnt)])

  @pl.when(jnp.logical_and(cid == 0, t < NS - 1))
  def _():
    eq_reduce(t * 640, 640)

  @pl.when(jnp.logical_and(cid == 0, t == NS - 1))
  def _():
    eq_reduce((NS - 1) * 640, 400)


def _sc_scatter(zeros, msgl, msgr, eqxa, eqya, eqza, colp):
  k = functools.partial(
      pl.kernel,
      out_type=(
          jax.ShapeDtypeStruct((N_NODES, 128), jnp.float32),
          jax.ShapeDtypeStruct((N_NODES, 128), jnp.float32),
          jax.ShapeDtypeStruct((3 * N_NODES,), jnp.float32),
          jax.ShapeDtypeStruct((NS * 3 * N_NODES,), jnp.float32),
      ),
      mesh=_mesh(),
      scratch_types=[
          pltpu.VMEM((128,), jnp.int32),
          pltpu.VMEM((128,), jnp.int32),
          pltpu.VMEM((128, 128), jnp.float32),
          pltpu.VMEM((128,), jnp.float32),
          pltpu.VMEM((128,), jnp.float32),
          pltpu.VMEM((128,), jnp.float32),
          pltpu.VMEM((N_NODES,), jnp.float32),
          pltpu.VMEM((N_NODES,), jnp.float32),
          pltpu.VMEM((N_NODES,), jnp.float32),
          pltpu.VMEM((128, 128), jnp.float32),
          pltpu.VMEM((640,), jnp.float32),
          pltpu.VMEM_SHARED((ACC_ROWS, 128), jnp.float32),
      ],
      compiler_params=pltpu.CompilerParams(needs_layout_passes=False),
  )(_scatter_body)
  return k(zeros, msgl, msgr, eqxa, eqya, eqza, colp)


def kernel(pos, edge_index, node_type, edge_type, W_fourier, node_table,
           edge_table, in_W, in_b, dist_W, dist_b, proj_W1, proj_b1, proj_W2,
           proj_b2):
  f32 = jnp.float32
  pad_e = E_PAD - N_EDGES
  row = edge_index[0]
  col = edge_index[1]
  rowp = jnp.concatenate([row, jnp.zeros((pad_e,), row.dtype)])
  colp = jnp.concatenate([col, jnp.zeros((pad_e,), col.dtype)])
  posx, posy, posz = pos[:, 0], pos[:, 1], pos[:, 2]
  ntf = node_type.astype(f32)
  et_row = jnp.pad(edge_type.astype(f32), (0, pad_e)).reshape(1, E_PAD)

  ntp = jnp.concatenate([node_table, jnp.zeros((28, HID), f32)], axis=0)
  etab = jnp.concatenate([edge_table, jnp.zeros((28, HID), f32)], axis=0)
  w_cat = jnp.concatenate([in_W[:HID], in_W[HID:]], axis=1)
  w2pi = (W_fourier * (2.0 * jnp.pi)).reshape(1, HID)
  w1s, w1c = proj_W1[:HID], proj_W1[HID:2 * HID]
  w1ab = proj_W1[2 * HID:]
  b1 = proj_b1.reshape(1, HID)
  b2 = proj_b2.reshape(1, HID)
  inb = in_b.reshape(1, HID)
  db = dist_b.reshape(1, HID)

  ab = _tc_prep(ntp, w_cat)
  g_arr = _sc_gather(posx, posy, posz, ntf, rowp, colp)
  msgl, msgr, eqxa, eqya, eqza = _tc_main(g_arr, et_row, w2pi, w1s, w1c,
                                          w1ab, b1, proj_W2, b2, ab, inb,
                                          etab, dist_W, db)
  featl, featr, eqout, _ = _sc_scatter(jnp.zeros((128, 128), f32), msgl,
                                       msgr, eqxa, eqya, eqza, colp)
  return jnp.concatenate([featl, featr, eqout.reshape(3, N_NODES).T], axis=1)


# speedup vs baseline: 3.2540x; 1.1358x over previous
import functools

import jax
import jax.numpy as jnp
from jax import lax
from jax.experimental import pallas as pl
from jax.experimental.pallas import tpu as pltpu
from jax.experimental.pallas import tpu_sc as plsc

N_NODES = 10000
N_EDGES = 160000
E_PAD = 163840
HID = 256
NC, NS = 2, 16
PER_W = E_PAD // (NC * NS)
CK = 1024
EB = 512
PER_T = E_PAD // NS
ROWS_PER_T = 625

_mesh = lambda: plsc.VectorSubcoreMesh(
    core_axis_name="c", subcore_axis_name="s", num_cores=NC, num_subcores=NS)


def _gather_body(posx, posy, posz, ntf, rowp, colp, g_out,
                 ridx, cidx, gbuf, sem):
  wid = lax.axis_index("s") * NC + lax.axis_index("c")
  for ch in range(PER_W // CK):
    base = wid * PER_W + ch * CK
    pltpu.sync_copy(rowp.at[pl.ds(base, CK)], ridx)
    pltpu.sync_copy(colp.at[pl.ds(base, CK)], cidx)
    descs = []
    for k in range(CK // 128):
      isl = pl.ds(k * 128, 128)
      for c, tab in enumerate((posx, posy, posz)):
        descs.append(
            pltpu.async_copy(tab.at[ridx.at[isl]], gbuf.at[c, isl], sem))
        descs.append(
            pltpu.async_copy(tab.at[cidx.at[isl]], gbuf.at[3 + c, isl], sem))
      descs.append(
          pltpu.async_copy(ntf.at[ridx.at[isl]], gbuf.at[6, isl], sem))
      descs.append(
          pltpu.async_copy(ntf.at[cidx.at[isl]], gbuf.at[7, isl], sem))
    for dsc in descs:
      dsc.wait()
    pltpu.sync_copy(gbuf, g_out.at[:, pl.ds(base, CK)])


def _sc_gather(posx, posy, posz, ntf, rowp, colp):
  k = functools.partial(
      pl.kernel,
      out_type=jax.ShapeDtypeStruct((8, E_PAD), jnp.float32),
      mesh=_mesh(),
      scratch_types=[
          pltpu.VMEM((CK,), jnp.int32),
          pltpu.VMEM((CK,), jnp.int32),
          pltpu.VMEM((8, CK), jnp.float32),
          pltpu.SemaphoreType.DMA,
      ],
      compiler_params=pltpu.CompilerParams(needs_layout_passes=False),
  )(_gather_body)
  return k(posx, posy, posz, ntf, rowp, colp)


def _prep_body(nt_ref, w_ref, o_ref):
  o_ref[...] = jnp.dot(nt_ref[...], w_ref[...],
                       preferred_element_type=jnp.float32,
                       precision=jax.lax.Precision.HIGHEST)


def _tc_prep(ntp, w_cat):
  return pl.pallas_call(
      _prep_body,
      out_shape=jax.ShapeDtypeStruct((128, 2 * HID), jnp.float32),
  )(ntp, w_cat)


def _main_body(g_ref, et_ref, w2pi_ref, w1s_ref, w1c_ref, w1ab_ref, b1_ref,
               w2_ref, b2_ref, ab_ref, inb_ref, etab_ref, dw_ref, db_ref,
               msgl_ref, msgr_ref, eqx_ref, eqy_ref, eqz_ref):
  g = g_ref[...]
  prx, pry, prz = g[0:1, :], g[1:2, :], g[2:3, :]
  pcx, pcy, pcz = g[3:4, :], g[4:5, :], g[5:6, :]
  trow, tcol = g[6:7, :], g[7:8, :]
  et = et_ref[...]

  dx, dy, dz = prx - pcx, pry - pcy, prz - pcz
  radial = dx * dx + dy * dy + dz * dz
  d = jnp.sqrt(radial + 1e-4)
  norm = jnp.sqrt(radial + 1e-12) + 1.0
  dnx, dny, dnz = dx / norm, dy / norm, dz / norm
  cx = pry * pcz - prz * pcy
  cy = prz * pcx - prx * pcz
  cz = prx * pcy - pry * pcx
  cn = jnp.sqrt(cx * cx + cy * cy + cz * cz + 1e-12) + 1.0
  cnx, cny, cnz = cx / cn, cy / cn, cz / cn
  vx = dny * cnz - dnz * cny
  vy = dnz * cnx - dnx * cnz
  vz = dnx * cny - dny * cnx
  nr = jnp.sqrt(prx * prx + pry * pry + prz * prz) + 1e-5
  ncn = jnp.sqrt(pcx * pcx + pcy * pcy + pcz * pcz) + 1e-5
  cos_t = (prx * pcx + pry * pcy + prz * pcz) / (nr * ncn)
  sin_t = jnp.sqrt(jnp.clip(1.0 - cos_t * cos_t, 0.0, 1.0))

  eqx_ref[...] = (dnx + vx).reshape((EB,))
  eqy_ref[...] = (dny + vy).reshape((EB,))
  eqz_ref[...] = (dnz + vz).reshape((EB,))

  scal = jnp.concatenate(
      [d, cos_t, sin_t, trow, tcol, et, jnp.zeros((2, EB), jnp.float32)],
      axis=0)
  st = scal.T
  d_c = st[:, 0:1]
  angles = st[:, 1:3]
  trow_c, tcol_c, et_c = st[:, 3:4], st[:, 4:5], st[:, 5:6]

  f32 = jnp.float32
  dot = functools.partial(jax.lax.dot_general, preferred_element_type=f32,
                          precision=jax.lax.Precision.DEFAULT)
  dotx = functools.partial(jax.lax.dot_general, preferred_element_type=f32,
                           precision=jax.lax.Precision.HIGHEST)
  nn = (((1,), (0,)), ((), ()))

  x = dotx(d_c, w2pi_ref[...], nn)
  n = jnp.round(x * 0.3183098861837907)
  r = x - n * 3.140625 - n * 9.67653589793e-4
  odd = jnp.bitwise_and(n.astype(jnp.int32), 1)
  sgn = jnp.where(odd == 1, -1.0, 1.0)
  r2 = r * r
  sp = r * (1.0 + r2 * (-1.6666654611e-1
                        + r2 * (8.3321608736e-3 + r2 * -1.9515295891e-4)))
  cp = (1.0 - 0.5 * r2
        + r2 * r2 * (4.166664568298827e-2
                     + r2 * (-1.388731625493765e-3
                             + r2 * 2.443315711809948e-5)))
  p1 = (dot(sgn * sp, w1s_ref[...], nn)
        + dot(sgn * cp, w1c_ref[...], nn)
        + dotx(angles, w1ab_ref[...], nn)
        + b1_ref[...])
  p = dot(jnp.maximum(p1, 0.0), w2_ref[...], nn) + b2_ref[...]

  lane = lax.broadcasted_iota(jnp.int32, (EB, 128), 1).astype(f32)
  ohr = (lane == trow_c).astype(f32)
  ohc = (lane == tcol_c).astype(f32)
  ohe = (lane == et_c).astype(f32)
  ab = ab_ref[...]
  he = jnp.maximum(
      dot(ohr, ab[:, :HID], nn) + dot(ohc, ab[:, HID:], nn) + inb_ref[...],
      0.0)
  ea = dot(ohe, etab_ref[...], nn)
  dml = jnp.maximum(dotx(d_c, dw_ref[...], nn) + db_ref[...], 0.0)

  msg = he * ea * dml + p
  msgl_ref[...] = msg[:, :128]
  msgr_ref[...] = msg[:, 128:]


E_HALF = E_PAD // 2


def _tc_main(g_arr, et_row, w2pi, w1s, w1c, w1ab, b1, w2, b2, ab, inb, etab,
             dw, db, off_blk):
  nblk = E_HALF // EB
  full = lambda shape: pl.BlockSpec(shape, lambda i: (0,) * len(shape))
  return pl.pallas_call(
      _main_body,
      grid=(nblk,),
      in_specs=[
          pl.BlockSpec((8, EB), lambda i: (0, i + off_blk)),
          pl.BlockSpec((1, EB), lambda i: (0, i + off_blk)),
          full((1, HID)), full((HID, HID)), full((HID, HID)),
          full((2, HID)), full((1, HID)), full((HID, HID)), full((1, HID)),
          full((128, 2 * HID)), full((1, HID)), full((128, HID)),
          full((1, HID)), full((1, HID)),
      ],
      out_specs=[
          pl.BlockSpec((EB, 128), lambda i: (i, 0)),
          pl.BlockSpec((EB, 128), lambda i: (i, 0)),
          pl.BlockSpec((EB,), lambda i: (i,)),
          pl.BlockSpec((EB,), lambda i: (i,)),
          pl.BlockSpec((EB,), lambda i: (i,)),
      ],
      out_shape=[
          jax.ShapeDtypeStruct((E_HALF, 128), jnp.float32),
          jax.ShapeDtypeStruct((E_HALF, 128), jnp.float32),
          jax.ShapeDtypeStruct((E_HALF,), jnp.float32),
          jax.ShapeDtypeStruct((E_HALF,), jnp.float32),
          jax.ShapeDtypeStruct((E_HALF,), jnp.float32),
      ],
      compiler_params=pltpu.CompilerParams(
          dimension_semantics=("arbitrary",)),
  )(g_arr, et_row, w2pi, w1s, w1c, w1ab, b1, w2, b2, ab, inb, etab, dw, db)


HALF_N = N_NODES // 2
ACC_ROWS = 6144
W_ROWS = 312
PER_T2 = E_HALF // NS


def _make_scatter_body(off_e, has_prev):
  def _scatter_body(zeros_hbm, msgl, msgr, eqxa, eqya, eqza, colp,
                    pfl, pfr, peq,
                    featl, featr, eqout, speq,
                    idxv, idxw, rows, eqb0, eqb1, eqb2, eqa0, eqa1, eqa2,
                    zsrc, eqred, acc):
    cid = lax.axis_index("c")
    t = lax.axis_index("s")

    pltpu.sync_copy(zeros_hbm, zsrc)

    def zeq(i, carry):
      z = jnp.zeros((16,), jnp.float32)
      eqa0[pl.ds(i * 16, 16)] = z
      eqa1[pl.ds(i * 16, 16)] = z
      eqa2[pl.ds(i * 16, 16)] = z
      return carry
    lax.fori_loop(0, N_NODES // 16, zeq, 0)

    start = t * PER_T2
    n_ch = jnp.maximum(
        jnp.minimum(N_EDGES - off_e - start, PER_T2), 0) // 128

    def run(msgx, h, with_eq):
      def chunk(j, carry):
        base = start + j * 128
        pltpu.sync_copy(colp.at[pl.ds(off_e + base, 128)], idxv)
        pltpu.sync_copy(msgx.at[pl.ds(base, 128), :], rows)
        if with_eq:
          pltpu.sync_copy(eqxa.at[pl.ds(base, 128)], eqb0)
          pltpu.sync_copy(eqya.at[pl.ds(base, 128)], eqb1)
          pltpu.sync_copy(eqza.at[pl.ds(base, 128)], eqb2)
        for k in range(8):
          sl = pl.ds(k * 16, 16)
          c16 = idxv[sl]
          local = c16 - h * HALF_N
          valid = jnp.logical_and(local >= 0, local < HALF_N)
          idxw[sl] = jnp.where(valid, local,
                               HALF_N + jnp.bitwise_and(c16, 1023))
          if with_eq:
            plsc.addupdate_scatter(eqa0, [c16], eqb0[sl])
            plsc.addupdate_scatter(eqa1, [c16], eqb1[sl])
            plsc.addupdate_scatter(eqa2, [c16], eqb2[sl])
        pltpu.sync_copy(rows, acc.at[idxw], add=True)
        return carry
      lax.fori_loop(0, n_ch, chunk, 0)

    for h in range(2):
      def init_zero(rlo, n128s, rem=0):
        for j in range(n128s):
          pltpu.sync_copy(zsrc, acc.at[pl.ds(rlo + j * 128, 128)])
        if rem:
          pltpu.sync_copy(zsrc.at[pl.ds(0, rem)],
                          acc.at[pl.ds(rlo + n128s * 128, rem)])

      if not has_prev:
        init_zero(t * 384, 3)
      else:
        def init_prev(px):
          @pl.when(t < 13)
          def _():
            for j in range(3):
              pltpu.sync_copy(
                  px.at[pl.ds(h * HALF_N + t * 384 + j * 128, 128)],
                  acc.at[pl.ds(t * 384 + j * 128, 128)])
          @pl.when(t == 13)
          def _():
            pltpu.sync_copy(px.at[pl.ds(h * HALF_N + 4992, 8)],
                            acc.at[pl.ds(4992, 8)])
            init_zero(5000, 2, 120)
          @pl.when(t > 13)
          def _():
            init_zero(t * 384, 3)
        @pl.when(cid == 0)
        def _():
          init_prev(pfl)
        @pl.when(cid == 1)
        def _():
          init_prev(pfr)
      plsc.subcore_barrier()

      @pl.when(cid == 0)
      def _():
        run(msgl, h, with_eq=(h == 0))
        if h == 0:
          pltpu.sync_copy(eqa0, speq.at[pl.ds(t * 3 * N_NODES, N_NODES)])
          pltpu.sync_copy(eqa1,
                          speq.at[pl.ds(t * 3 * N_NODES + N_NODES,
                                        N_NODES)])
          pltpu.sync_copy(eqa2,
                          speq.at[pl.ds(t * 3 * N_NODES + 2 * N_NODES,
                                        N_NODES)])

      @pl.when(cid == 1)
      def _():
        run(msgr, h, with_eq=False)

      plsc.subcore_barrier()

      def writeout(featx, rstart, nrows):
        pltpu.sync_copy(acc.at[pl.ds(rstart, nrows)],
                        featx.at[pl.ds(h * HALF_N + rstart, nrows)])

      @pl.when(jnp.logical_and(cid == 0, t < NS - 1))
      def _():
        writeout(featl, t * W_ROWS, W_ROWS)
      @pl.when(jnp.logical_and(cid == 0, t == NS - 1))
      def _():
        writeout(featl, (NS - 1) * W_ROWS, HALF_N - (NS - 1) * W_ROWS)
      @pl.when(jnp.logical_and(cid == 1, t < NS - 1))
      def _():
        writeout(featr, t * W_ROWS, W_ROWS)
      @pl.when(jnp.logical_and(cid == 1, t == NS - 1))
      def _():
        writeout(featr, (NS - 1) * W_ROWS, HALF_N - (NS - 1) * W_ROWS)
      plsc.subcore_barrier()

    def eq_reduce(nbase, cnt):
      for r, eqa in ((0, eqa0), (1, eqa1), (2, eqa2)):
        def addv(i, carry):
          sl16 = pl.ds(i * 16, 16)
          eqa[sl16] = eqa[sl16] + eqred[sl16]
          return carry

        if has_prev:
          pltpu.sync_copy(peq.at[pl.ds(r * N_NODES + nbase, cnt)],
                          eqa.at[pl.ds(0, cnt)])
          srcs = range(NS)
        else:
          pltpu.sync_copy(speq.at[pl.ds(r * N_NODES + nbase, cnt)],
                          eqa.at[pl.ds(0, cnt)])
          srcs = range(1, NS)
        for src in srcs:
          pltpu.sync_copy(
              speq.at[pl.ds((src * 3 + r) * N_NODES + nbase, cnt)],
              eqred.at[pl.ds(0, cnt)])
          lax.fori_loop(0, cnt // 16, addv, 0)
        pltpu.sync_copy(eqa.at[pl.ds(0, cnt)],
                        eqout.at[pl.ds(r * N_NODES + nbase, cnt)])

    @pl.when(jnp.logical_and(cid == 0, t < NS - 1))
    def _():
      eq_reduce(t * 640, 640)

    @pl.when(jnp.logical_and(cid == 0, t == NS - 1))
    def _():
      eq_reduce((NS - 1) * 640, 400)

  return _scatter_body


def _sc_scatter(zeros, msgl, msgr, eqxa, eqya, eqza, colp, pfl, pfr, peq,
                off_e, has_prev):
  k = functools.partial(
      pl.kernel,
      out_type=(
          jax.ShapeDtypeStruct((N_NODES, 128), jnp.float32),
          jax.ShapeDtypeStruct((N_NODES, 128), jnp.float32),
          jax.ShapeDtypeStruct((3 * N_NODES,), jnp.float32),
          jax.ShapeDtypeStruct((NS * 3 * N_NODES,), jnp.float32),
      ),
      mesh=_mesh(),
      scratch_types=[
          pltpu.VMEM((128,), jnp.int32),
          pltpu.VMEM((128,), jnp.int32),
          pltpu.VMEM((128, 128), jnp.float32),
          pltpu.VMEM((128,), jnp.float32),
          pltpu.VMEM((128,), jnp.float32),
          pltpu.VMEM((128,), jnp.float32),
          pltpu.VMEM((N_NODES,), jnp.float32),
          pltpu.VMEM((N_NODES,), jnp.float32),
          pltpu.VMEM((N_NODES,), jnp.float32),
          pltpu.VMEM((128, 128), jnp.float32),
          pltpu.VMEM((640,), jnp.float32),
          pltpu.VMEM_SHARED((ACC_ROWS, 128), jnp.float32),
      ],
      compiler_params=pltpu.CompilerParams(needs_layout_passes=False),
  )(_make_scatter_body(off_e, has_prev))
  return k(zeros, msgl, msgr, eqxa, eqya, eqza, colp, pfl, pfr, peq)


def kernel(pos, edge_index, node_type, edge_type, W_fourier, node_table,
           edge_table, in_W, in_b, dist_W, dist_b, proj_W1, proj_b1, proj_W2,
           proj_b2):
  f32 = jnp.float32
  pad_e = E_PAD - N_EDGES
  row = edge_index[0]
  col = edge_index[1]
  rowp = jnp.concatenate([row, jnp.zeros((pad_e,), row.dtype)])
  colp = jnp.concatenate([col, jnp.zeros((pad_e,), col.dtype)])
  posx, posy, posz = pos[:, 0], pos[:, 1], pos[:, 2]
  ntf = node_type.astype(f32)
  et_row = jnp.pad(edge_type.astype(f32), (0, pad_e)).reshape(1, E_PAD)

  ntp = jnp.concatenate([node_table, jnp.zeros((28, HID), f32)], axis=0)
  etab = jnp.concatenate([edge_table, jnp.zeros((28, HID), f32)], axis=0)
  w_cat = jnp.concatenate([in_W[:HID], in_W[HID:]], axis=1)
  w2pi = (W_fourier * (2.0 * jnp.pi)).reshape(1, HID)
  w1s, w1c = proj_W1[:HID], proj_W1[HID:2 * HID]
  w1ab = proj_W1[2 * HID:]
  b1 = proj_b1.reshape(1, HID)
  b2 = proj_b2.reshape(1, HID)
  inb = in_b.reshape(1, HID)
  db = dist_b.reshape(1, HID)

  ab = _tc_prep(ntp, w_cat)
  g_arr = _sc_gather(posx, posy, posz, ntf, rowp, colp)
  zeros = jnp.zeros((128, 128), f32)
  m1 = _tc_main(g_arr, et_row, w2pi, w1s, w1c, w1ab, b1, proj_W2, b2, ab,
                inb, etab, dist_W, db, 0)
  m2 = _tc_main(g_arr, et_row, w2pi, w1s, w1c, w1ab, b1, proj_W2, b2, ab,
                inb, etab, dist_W, db, E_HALF // EB)
  fl1, fr1, eq1, sp1 = _sc_scatter(zeros, *m1, colp, zeros, zeros, zeros,
                                   0, False)
  featl, featr, eqout, _ = _sc_scatter(zeros, *m2, colp, fl1, fr1, eq1,
                                       E_HALF, True)
  return jnp.concatenate([featl, featr, eqout.reshape(3, N_NODES).T], axis=1)


# speedup vs baseline: 3.2556x; 1.0005x over previous
import functools

import jax
import jax.numpy as jnp
from jax import lax
from jax.experimental import pallas as pl
from jax.experimental.pallas import tpu as pltpu
from jax.experimental.pallas import tpu_sc as plsc

N_NODES = 10000
N_EDGES = 160000
E_PAD = 163840
HID = 256
NC, NS = 2, 16
PER_W = E_PAD // (NC * NS)
CK = 1024
EB = 512
PER_T = E_PAD // NS
ROWS_PER_T = 625

_mesh = lambda: plsc.VectorSubcoreMesh(
    core_axis_name="c", subcore_axis_name="s", num_cores=NC, num_subcores=NS)


def _gather_body(posx, posy, posz, ntf, rowp, colp, g_out,
                 ridx, cidx, gbuf, sem):
  wid = lax.axis_index("s") * NC + lax.axis_index("c")
  for ch in range(PER_W // CK):
    base = wid * PER_W + ch * CK
    pltpu.sync_copy(rowp.at[pl.ds(base, CK)], ridx)
    pltpu.sync_copy(colp.at[pl.ds(base, CK)], cidx)
    descs = []
    for k in range(CK // 128):
      isl = pl.ds(k * 128, 128)
      for c, tab in enumerate((posx, posy, posz)):
        descs.append(
            pltpu.async_copy(tab.at[ridx.at[isl]], gbuf.at[c, isl], sem))
        descs.append(
            pltpu.async_copy(tab.at[cidx.at[isl]], gbuf.at[3 + c, isl], sem))
      descs.append(
          pltpu.async_copy(ntf.at[ridx.at[isl]], gbuf.at[6, isl], sem))
      descs.append(
          pltpu.async_copy(ntf.at[cidx.at[isl]], gbuf.at[7, isl], sem))
    for dsc in descs:
      dsc.wait()
    pltpu.sync_copy(gbuf, g_out.at[:, pl.ds(base, CK)])


def _sc_gather(posx, posy, posz, ntf, rowp, colp):
  k = functools.partial(
      pl.kernel,
      out_type=jax.ShapeDtypeStruct((8, E_PAD), jnp.float32),
      mesh=_mesh(),
      scratch_types=[
          pltpu.VMEM((CK,), jnp.int32),
          pltpu.VMEM((CK,), jnp.int32),
          pltpu.VMEM((8, CK), jnp.float32),
          pltpu.SemaphoreType.DMA,
      ],
      compiler_params=pltpu.CompilerParams(needs_layout_passes=False),
  )(_gather_body)
  return k(posx, posy, posz, ntf, rowp, colp)


def _prep_body(nt_ref, w_ref, o_ref):
  o_ref[...] = jnp.dot(nt_ref[...], w_ref[...],
                       preferred_element_type=jnp.float32,
                       precision=jax.lax.Precision.HIGHEST)


def _tc_prep(ntp, w_cat):
  return pl.pallas_call(
      _prep_body,
      out_shape=jax.ShapeDtypeStruct((128, 2 * HID), jnp.float32),
  )(ntp, w_cat)


def _main_body(g_ref, et_ref, w2pi_ref, w1s_ref, w1c_ref, w1ab_ref, b1_ref,
               w2_ref, b2_ref, ab_ref, inb_ref, etab_ref, dw_ref, db_ref,
               msgl_ref, msgr_ref, eqx_ref, eqy_ref, eqz_ref):
  g = g_ref[...]
  prx, pry, prz = g[0:1, :], g[1:2, :], g[2:3, :]
  pcx, pcy, pcz = g[3:4, :], g[4:5, :], g[5:6, :]
  trow, tcol = g[6:7, :], g[7:8, :]
  et = et_ref[...]

  dx, dy, dz = prx - pcx, pry - pcy, prz - pcz
  radial = dx * dx + dy * dy + dz * dz
  d = jnp.sqrt(radial + 1e-4)
  norm = jnp.sqrt(radial + 1e-12) + 1.0
  dnx, dny, dnz = dx / norm, dy / norm, dz / norm
  cx = pry * pcz - prz * pcy
  cy = prz * pcx - prx * pcz
  cz = prx * pcy - pry * pcx
  cn = jnp.sqrt(cx * cx + cy * cy + cz * cz + 1e-12) + 1.0
  cnx, cny, cnz = cx / cn, cy / cn, cz / cn
  vx = dny * cnz - dnz * cny
  vy = dnz * cnx - dnx * cnz
  vz = dnx * cny - dny * cnx
  nr = jnp.sqrt(prx * prx + pry * pry + prz * prz) + 1e-5
  ncn = jnp.sqrt(pcx * pcx + pcy * pcy + pcz * pcz) + 1e-5
  cos_t = (prx * pcx + pry * pcy + prz * pcz) / (nr * ncn)
  sin_t = jnp.sqrt(jnp.clip(1.0 - cos_t * cos_t, 0.0, 1.0))

  eqx_ref[...] = (dnx + vx).reshape((EB,))
  eqy_ref[...] = (dny + vy).reshape((EB,))
  eqz_ref[...] = (dnz + vz).reshape((EB,))

  scal = jnp.concatenate(
      [d, cos_t, sin_t, trow, tcol, et, jnp.zeros((2, EB), jnp.float32)],
      axis=0)
  st = scal.T
  d_c = st[:, 0:1]
  angles = st[:, 1:3]
  trow_c, tcol_c, et_c = st[:, 3:4], st[:, 4:5], st[:, 5:6]

  f32 = jnp.float32
  dot = functools.partial(jax.lax.dot_general, preferred_element_type=f32,
                          precision=jax.lax.Precision.DEFAULT)
  dotx = functools.partial(jax.lax.dot_general, preferred_element_type=f32,
                           precision=jax.lax.Precision.HIGHEST)
  nn = (((1,), (0,)), ((), ()))

  x = dotx(d_c, w2pi_ref[...], nn)
  n = jnp.round(x * 0.3183098861837907)
  r = x - n * 3.140625 - n * 9.67653589793e-4
  odd = jnp.bitwise_and(n.astype(jnp.int32), 1)
  sgn = jnp.where(odd == 1, -1.0, 1.0)
  r2 = r * r
  sp = r * (1.0 + r2 * (-1.6666654611e-1
                        + r2 * (8.3321608736e-3 + r2 * -1.9515295891e-4)))
  cp = (1.0 - 0.5 * r2
        + r2 * r2 * (4.166664568298827e-2
                     + r2 * (-1.388731625493765e-3
                             + r2 * 2.443315711809948e-5)))
  bf16 = jnp.bfloat16
  p1 = (dot((sgn * sp).astype(bf16), w1s_ref[...], nn)
        + dot((sgn * cp).astype(bf16), w1c_ref[...], nn)
        + dotx(angles, w1ab_ref[...], nn)
        + b1_ref[...])
  p = dot(jnp.maximum(p1, 0.0).astype(bf16), w2_ref[...], nn) + b2_ref[...]

  lane = lax.broadcasted_iota(jnp.int32, (EB, 128), 1).astype(f32)
  ohr = (lane == trow_c).astype(f32)
  ohc = (lane == tcol_c).astype(f32)
  ohe = (lane == et_c).astype(f32)
  ab = ab_ref[...]
  he = jnp.maximum(
      dot(ohr, ab[:, :HID], nn) + dot(ohc, ab[:, HID:], nn) + inb_ref[...],
      0.0)
  ea = dot(ohe, etab_ref[...], nn)
  dml = jnp.maximum(dotx(d_c, dw_ref[...], nn) + db_ref[...], 0.0)

  msg = he * ea * dml + p
  msgl_ref[...] = msg[:, :128]
  msgr_ref[...] = msg[:, 128:]


E_HALF = E_PAD // 2


def _tc_main(g_arr, et_row, w2pi, w1s, w1c, w1ab, b1, w2, b2, ab, inb, etab,
             dw, db, off_blk):
  nblk = E_HALF // EB
  full = lambda shape: pl.BlockSpec(shape, lambda i: (0,) * len(shape))
  return pl.pallas_call(
      _main_body,
      grid=(nblk,),
      in_specs=[
          pl.BlockSpec((8, EB), lambda i: (0, i + off_blk)),
          pl.BlockSpec((1, EB), lambda i: (0, i + off_blk)),
          full((1, HID)), full((HID, HID)), full((HID, HID)),
          full((2, HID)), full((1, HID)), full((HID, HID)), full((1, HID)),
          full((128, 2 * HID)), full((1, HID)), full((128, HID)),
          full((1, HID)), full((1, HID)),
      ],
      out_specs=[
          pl.BlockSpec((EB, 128), lambda i: (i, 0)),
          pl.BlockSpec((EB, 128), lambda i: (i, 0)),
          pl.BlockSpec((EB,), lambda i: (i,)),
          pl.BlockSpec((EB,), lambda i: (i,)),
          pl.BlockSpec((EB,), lambda i: (i,)),
      ],
      out_shape=[
          jax.ShapeDtypeStruct((E_HALF, 128), jnp.float32),
          jax.ShapeDtypeStruct((E_HALF, 128), jnp.float32),
          jax.ShapeDtypeStruct((E_HALF,), jnp.float32),
          jax.ShapeDtypeStruct((E_HALF,), jnp.float32),
          jax.ShapeDtypeStruct((E_HALF,), jnp.float32),
      ],
      compiler_params=pltpu.CompilerParams(
          dimension_semantics=("arbitrary",)),
  )(g_arr, et_row, w2pi, w1s, w1c, w1ab, b1, w2, b2, ab, inb, etab, dw, db)


HALF_N = N_NODES // 2
ACC_ROWS = 6144
W_ROWS = 312
PER_T2 = E_HALF // NS


def _make_scatter_body(off_e, has_prev):
  def _scatter_body(zeros_hbm, msgl, msgr, eqxa, eqya, eqza, colp,
                    pfl, pfr, peq,
                    featl, featr, eqout, speq,
                    idxv, idxw, rows, eqb0, eqb1, eqb2, eqa0, eqa1, eqa2,
                    zsrc, eqred, acc):
    cid = lax.axis_index("c")
    t = lax.axis_index("s")

    pltpu.sync_copy(zeros_hbm, zsrc)

    def zeq(i, carry):
      z = jnp.zeros((16,), jnp.float32)
      eqa0[pl.ds(i * 16, 16)] = z
      eqa1[pl.ds(i * 16, 16)] = z
      eqa2[pl.ds(i * 16, 16)] = z
      return carry
    lax.fori_loop(0, N_NODES // 16, zeq, 0)

    start = t * PER_T2
    n_ch = jnp.maximum(
        jnp.minimum(N_EDGES - off_e - start, PER_T2), 0) // 128

    def run(msgx, h, with_eq):
      def chunk(j, carry):
        base = start + j * 128
        pltpu.sync_copy(colp.at[pl.ds(off_e + base, 128)], idxv)
        pltpu.sync_copy(msgx.at[pl.ds(base, 128), :], rows)
        if with_eq:
          pltpu.sync_copy(eqxa.at[pl.ds(base, 128)], eqb0)
          pltpu.sync_copy(eqya.at[pl.ds(base, 128)], eqb1)
          pltpu.sync_copy(eqza.at[pl.ds(base, 128)], eqb2)
        for k in range(8):
          sl = pl.ds(k * 16, 16)
          c16 = idxv[sl]
          local = c16 - h * HALF_N
          valid = jnp.logical_and(local >= 0, local < HALF_N)
          idxw[sl] = jnp.where(valid, local,
                               HALF_N + jnp.bitwise_and(c16, 1023))
          if with_eq:
            plsc.addupdate_scatter(eqa0, [c16], eqb0[sl])
            plsc.addupdate_scatter(eqa1, [c16], eqb1[sl])
            plsc.addupdate_scatter(eqa2, [c16], eqb2[sl])
        pltpu.sync_copy(rows, acc.at[idxw], add=True)
        return carry
      lax.fori_loop(0, n_ch, chunk, 0)

    for h in range(2):
      def init_zero(rlo, n128s, rem=0):
        for j in range(n128s):
          pltpu.sync_copy(zsrc, acc.at[pl.ds(rlo + j * 128, 128)])
        if rem:
          pltpu.sync_copy(zsrc.at[pl.ds(0, rem)],
                          acc.at[pl.ds(rlo + n128s * 128, rem)])

      if not has_prev:
        init_zero(t * 384, 3)
      else:
        def init_prev(px):
          @pl.when(t < 13)
          def _():
            for j in range(3):
              pltpu.sync_copy(
                  px.at[pl.ds(h * HALF_N + t * 384 + j * 128, 128)],
                  acc.at[pl.ds(t * 384 + j * 128, 128)])
          @pl.when(t == 13)
          def _():
            pltpu.sync_copy(px.at[pl.ds(h * HALF_N + 4992, 8)],
                            acc.at[pl.ds(4992, 8)])
            init_zero(5000, 2, 120)
          @pl.when(t > 13)
          def _():
            init_zero(t * 384, 3)
        @pl.when(cid == 0)
        def _():
          init_prev(pfl)
        @pl.when(cid == 1)
        def _():
          init_prev(pfr)
      plsc.subcore_barrier()

      @pl.when(cid == 0)
      def _():
        run(msgl, h, with_eq=(h == 0))
        if h == 0:
          pltpu.sync_copy(eqa0, speq.at[pl.ds(t * 3 * N_NODES, N_NODES)])
          pltpu.sync_copy(eqa1,
                          speq.at[pl.ds(t * 3 * N_NODES + N_NODES,
                                        N_NODES)])
          pltpu.sync_copy(eqa2,
                          speq.at[pl.ds(t * 3 * N_NODES + 2 * N_NODES,
                                        N_NODES)])

      @pl.when(cid == 1)
      def _():
        run(msgr, h, with_eq=False)

      plsc.subcore_barrier()

      def writeout(featx, rstart, nrows):
        pltpu.sync_copy(acc.at[pl.ds(rstart, nrows)],
                        featx.at[pl.ds(h * HALF_N + rstart, nrows)])

      @pl.when(jnp.logical_and(cid == 0, t < NS - 1))
      def _():
        writeout(featl, t * W_ROWS, W_ROWS)
      @pl.when(jnp.logical_and(cid == 0, t == NS - 1))
      def _():
        writeout(featl, (NS - 1) * W_ROWS, HALF_N - (NS - 1) * W_ROWS)
      @pl.when(jnp.logical_and(cid == 1, t < NS - 1))
      def _():
        writeout(featr, t * W_ROWS, W_ROWS)
      @pl.when(jnp.logical_and(cid == 1, t == NS - 1))
      def _():
        writeout(featr, (NS - 1) * W_ROWS, HALF_N - (NS - 1) * W_ROWS)
      plsc.subcore_barrier()

    def eq_reduce(nbase, cnt):
      for r, eqa in ((0, eqa0), (1, eqa1), (2, eqa2)):
        def addv(i, carry):
          sl16 = pl.ds(i * 16, 16)
          eqa[sl16] = eqa[sl16] + eqred[sl16]
          return carry

        if has_prev:
          pltpu.sync_copy(peq.at[pl.ds(r * N_NODES + nbase, cnt)],
                          eqa.at[pl.ds(0, cnt)])
          srcs = range(NS)
        else:
          pltpu.sync_copy(speq.at[pl.ds(r * N_NODES + nbase, cnt)],
                          eqa.at[pl.ds(0, cnt)])
          srcs = range(1, NS)
        for src in srcs:
          pltpu.sync_copy(
              speq.at[pl.ds((src * 3 + r) * N_NODES + nbase, cnt)],
              eqred.at[pl.ds(0, cnt)])
          lax.fori_loop(0, cnt // 16, addv, 0)
        pltpu.sync_copy(eqa.at[pl.ds(0, cnt)],
                        eqout.at[pl.ds(r * N_NODES + nbase, cnt)])

    @pl.when(jnp.logical_and(cid == 0, t < NS - 1))
    def _():
      eq_reduce(t * 640, 640)

    @pl.when(jnp.logical_and(cid == 0, t == NS - 1))
    def _():
      eq_reduce((NS - 1) * 640, 400)

  return _scatter_body


def _sc_scatter(zeros, msgl, msgr, eqxa, eqya, eqza, colp, pfl, pfr, peq,
                off_e, has_prev):
  k = functools.partial(
      pl.kernel,
      out_type=(
          jax.ShapeDtypeStruct((N_NODES, 128), jnp.float32),
          jax.ShapeDtypeStruct((N_NODES, 128), jnp.float32),
          jax.ShapeDtypeStruct((3 * N_NODES,), jnp.float32),
          jax.ShapeDtypeStruct((NS * 3 * N_NODES,), jnp.float32),
      ),
      mesh=_mesh(),
      scratch_types=[
          pltpu.VMEM((128,), jnp.int32),
          pltpu.VMEM((128,), jnp.int32),
          pltpu.VMEM((128, 128), jnp.float32),
          pltpu.VMEM((128,), jnp.float32),
          pltpu.VMEM((128,), jnp.float32),
          pltpu.VMEM((128,), jnp.float32),
          pltpu.VMEM((N_NODES,), jnp.float32),
          pltpu.VMEM((N_NODES,), jnp.float32),
          pltpu.VMEM((N_NODES,), jnp.float32),
          pltpu.VMEM((128, 128), jnp.float32),
          pltpu.VMEM((640,), jnp.float32),
          pltpu.VMEM_SHARED((ACC_ROWS, 128), jnp.float32),
      ],
      compiler_params=pltpu.CompilerParams(needs_layout_passes=False),
  )(_make_scatter_body(off_e, has_prev))
  return k(zeros, msgl, msgr, eqxa, eqya, eqza, colp, pfl, pfr, peq)


def kernel(pos, edge_index, node_type, edge_type, W_fourier, node_table,
           edge_table, in_W, in_b, dist_W, dist_b, proj_W1, proj_b1, proj_W2,
           proj_b2):
  f32 = jnp.float32
  pad_e = E_PAD - N_EDGES
  row = edge_index[0]
  col = edge_index[1]
  rowp = jnp.concatenate([row, jnp.zeros((pad_e,), row.dtype)])
  colp = jnp.concatenate([col, jnp.zeros((pad_e,), col.dtype)])
  posx, posy, posz = pos[:, 0], pos[:, 1], pos[:, 2]
  ntf = node_type.astype(f32)
  et_row = jnp.pad(edge_type.astype(f32), (0, pad_e)).reshape(1, E_PAD)

  ntp = jnp.concatenate([node_table, jnp.zeros((28, HID), f32)], axis=0)
  etab = jnp.concatenate([edge_table, jnp.zeros((28, HID), f32)], axis=0)
  w_cat = jnp.concatenate([in_W[:HID], in_W[HID:]], axis=1)
  w2pi = (W_fourier * (2.0 * jnp.pi)).reshape(1, HID)
  w1s = proj_W1[:HID].astype(jnp.bfloat16)
  w1c = proj_W1[HID:2 * HID].astype(jnp.bfloat16)
  w2b = proj_W2.astype(jnp.bfloat16)
  w1ab = proj_W1[2 * HID:]
  b1 = proj_b1.reshape(1, HID)
  b2 = proj_b2.reshape(1, HID)
  inb = in_b.reshape(1, HID)
  db = dist_b.reshape(1, HID)

  ab = _tc_prep(ntp, w_cat)
  g_arr = _sc_gather(posx, posy, posz, ntf, rowp, colp)
  zeros = jnp.zeros((128, 128), f32)
  m1 = _tc_main(g_arr, et_row, w2pi, w1s, w1c, w1ab, b1, w2b, b2, ab,
                inb, etab, dist_W, db, 0)
  m2 = _tc_main(g_arr, et_row, w2pi, w1s, w1c, w1ab, b1, w2b, b2, ab,
                inb, etab, dist_W, db, E_HALF // EB)
  fl1, fr1, eq1, sp1 = _sc_scatter(zeros, *m1, colp, zeros, zeros, zeros,
                                   0, False)
  featl, featr, eqout, _ = _sc_scatter(zeros, *m2, colp, fl1, fr1, eq1,
                                       E_HALF, True)
  return jnp.concatenate([featl, featr, eqout.reshape(3, N_NODES).T], axis=1)


# speedup vs baseline: 3.5282x; 1.0837x over previous
import functools

import jax
import jax.numpy as jnp
from jax import lax
from jax.experimental import pallas as pl
from jax.experimental.pallas import tpu as pltpu
from jax.experimental.pallas import tpu_sc as plsc

N_NODES = 10000
N_EDGES = 160000
E_PAD = 163840
HID = 256
NC, NS = 2, 16
PER_W = E_PAD // (NC * NS)
CK = 1024
EB = 512
PER_T = E_PAD // NS
ROWS_PER_T = 625

_mesh = lambda: plsc.VectorSubcoreMesh(
    core_axis_name="c", subcore_axis_name="s", num_cores=NC, num_subcores=NS)


def _gather_body(posx, posy, posz, ntf, rowp, colp, g_out,
                 ridx, cidx, gbuf, sem):
  wid = lax.axis_index("s") * NC + lax.axis_index("c")
  for ch in range(PER_W // CK):
    base = wid * PER_W + ch * CK
    pltpu.sync_copy(rowp.at[pl.ds(base, CK)], ridx)
    pltpu.sync_copy(colp.at[pl.ds(base, CK)], cidx)
    descs = []
    for k in range(CK // 128):
      isl = pl.ds(k * 128, 128)
      for c, tab in enumerate((posx, posy, posz)):
        descs.append(
            pltpu.async_copy(tab.at[ridx.at[isl]], gbuf.at[c, isl], sem))
        descs.append(
            pltpu.async_copy(tab.at[cidx.at[isl]], gbuf.at[3 + c, isl], sem))
      descs.append(
          pltpu.async_copy(ntf.at[ridx.at[isl]], gbuf.at[6, isl], sem))
      descs.append(
          pltpu.async_copy(ntf.at[cidx.at[isl]], gbuf.at[7, isl], sem))
    for dsc in descs:
      dsc.wait()
    pltpu.sync_copy(gbuf, g_out.at[:, pl.ds(base, CK)])


def _sc_gather(posx, posy, posz, ntf, rowp, colp):
  k = functools.partial(
      pl.kernel,
      out_type=jax.ShapeDtypeStruct((8, E_PAD), jnp.float32),
      mesh=_mesh(),
      scratch_types=[
          pltpu.VMEM((CK,), jnp.int32),
          pltpu.VMEM((CK,), jnp.int32),
          pltpu.VMEM((8, CK), jnp.float32),
          pltpu.SemaphoreType.DMA,
      ],
      compiler_params=pltpu.CompilerParams(needs_layout_passes=False),
  )(_gather_body)
  return k(posx, posy, posz, ntf, rowp, colp)


def _prep_body(nt_ref, w_ref, o_ref):
  o_ref[...] = jnp.dot(nt_ref[...], w_ref[...],
                       preferred_element_type=jnp.float32,
                       precision=jax.lax.Precision.HIGHEST)


def _tc_prep(ntp, w_cat):
  return pl.pallas_call(
      _prep_body,
      out_shape=jax.ShapeDtypeStruct((128, 2 * HID), jnp.float32),
  )(ntp, w_cat)


def _main_body(g_ref, et_ref, w2pi_ref, w1s_ref, w1c_ref, w1ab_ref, b1_ref,
               w2_ref, b2_ref, ab_ref, inb_ref, etab_ref, dw_ref, db_ref,
               msgl_ref, msgr_ref, eqx_ref, eqy_ref, eqz_ref):
  g = g_ref[...]
  prx, pry, prz = g[0:1, :], g[1:2, :], g[2:3, :]
  pcx, pcy, pcz = g[3:4, :], g[4:5, :], g[5:6, :]
  trow, tcol = g[6:7, :], g[7:8, :]
  et = et_ref[...]

  dx, dy, dz = prx - pcx, pry - pcy, prz - pcz
  radial = dx * dx + dy * dy + dz * dz
  d = jnp.sqrt(radial + 1e-4)
  norm = jnp.sqrt(radial + 1e-12) + 1.0
  dnx, dny, dnz = dx / norm, dy / norm, dz / norm
  cx = pry * pcz - prz * pcy
  cy = prz * pcx - prx * pcz
  cz = prx * pcy - pry * pcx
  cn = jnp.sqrt(cx * cx + cy * cy + cz * cz + 1e-12) + 1.0
  cnx, cny, cnz = cx / cn, cy / cn, cz / cn
  vx = dny * cnz - dnz * cny
  vy = dnz * cnx - dnx * cnz
  vz = dnx * cny - dny * cnx
  nr = jnp.sqrt(prx * prx + pry * pry + prz * prz) + 1e-5
  ncn = jnp.sqrt(pcx * pcx + pcy * pcy + pcz * pcz) + 1e-5
  cos_t = (prx * pcx + pry * pcy + prz * pcz) / (nr * ncn)
  sin_t = jnp.sqrt(jnp.clip(1.0 - cos_t * cos_t, 0.0, 1.0))

  eqx_ref[...] = (dnx + vx).reshape((EB,))
  eqy_ref[...] = (dny + vy).reshape((EB,))
  eqz_ref[...] = (dnz + vz).reshape((EB,))

  scal = jnp.concatenate(
      [d, cos_t, sin_t, trow, tcol, et, jnp.zeros((2, EB), jnp.float32)],
      axis=0)
  st = scal.T
  d_c = st[:, 0:1]
  angles = st[:, 1:3]
  trow_c, tcol_c, et_c = st[:, 3:4], st[:, 4:5], st[:, 5:6]

  f32 = jnp.float32
  dot = functools.partial(jax.lax.dot_general, preferred_element_type=f32,
                          precision=jax.lax.Precision.DEFAULT)
  dotx = functools.partial(jax.lax.dot_general, preferred_element_type=f32,
                           precision=jax.lax.Precision.HIGHEST)
  nn = (((1,), (0,)), ((), ()))

  x = dotx(d_c, w2pi_ref[...], nn)
  n = jnp.round(x * 0.3183098861837907)
  r = x - n * 3.140625 - n * 9.67653589793e-4
  odd = jnp.bitwise_and(n.astype(jnp.int32), 1)
  sgn = jnp.where(odd == 1, -1.0, 1.0)
  r2 = r * r
  sp = r * (1.0 + r2 * (-1.6666654611e-1
                        + r2 * (8.3321608736e-3 + r2 * -1.9515295891e-4)))
  cp = (1.0 - 0.5 * r2
        + r2 * r2 * (4.166664568298827e-2
                     + r2 * (-1.388731625493765e-3
                             + r2 * 2.443315711809948e-5)))
  bf16 = jnp.bfloat16
  p1 = (dot((sgn * sp).astype(bf16), w1s_ref[...], nn)
        + dot((sgn * cp).astype(bf16), w1c_ref[...], nn)
        + dotx(angles, w1ab_ref[...], nn)
        + b1_ref[...])
  p = dot(jnp.maximum(p1, 0.0).astype(bf16), w2_ref[...], nn) + b2_ref[...]

  lane = lax.broadcasted_iota(jnp.int32, (EB, 128), 1).astype(f32)
  ohr = (lane == trow_c).astype(f32)
  ohc = (lane == tcol_c).astype(f32)
  ohe = (lane == et_c).astype(f32)
  ab = ab_ref[...]
  he = jnp.maximum(
      dot(ohr, ab[:, :HID], nn) + dot(ohc, ab[:, HID:], nn) + inb_ref[...],
      0.0)
  ea = dot(ohe, etab_ref[...], nn)
  dml = jnp.maximum(dotx(d_c, dw_ref[...], nn) + db_ref[...], 0.0)

  msg = he * ea * dml + p
  msgl_ref[...] = msg[:, :128]
  msgr_ref[...] = msg[:, 128:]


E_HALF = E_PAD // 2


def _tc_main(g_arr, et_row, w2pi, w1s, w1c, w1ab, b1, w2, b2, ab, inb, etab,
             dw, db, off_blk):
  nblk = E_HALF // EB
  full = lambda shape: pl.BlockSpec(shape, lambda i: (0,) * len(shape))
  return pl.pallas_call(
      _main_body,
      grid=(nblk,),
      in_specs=[
          pl.BlockSpec((8, EB), lambda i: (0, i + off_blk)),
          pl.BlockSpec((1, EB), lambda i: (0, i + off_blk)),
          full((1, HID)), full((HID, HID)), full((HID, HID)),
          full((2, HID)), full((1, HID)), full((HID, HID)), full((1, HID)),
          full((128, 2 * HID)), full((1, HID)), full((128, HID)),
          full((1, HID)), full((1, HID)),
      ],
      out_specs=[
          pl.BlockSpec((EB, 128), lambda i: (i, 0)),
          pl.BlockSpec((EB, 128), lambda i: (i, 0)),
          pl.BlockSpec((EB,), lambda i: (i,)),
          pl.BlockSpec((EB,), lambda i: (i,)),
          pl.BlockSpec((EB,), lambda i: (i,)),
      ],
      out_shape=[
          jax.ShapeDtypeStruct((E_HALF, 128), jnp.float32),
          jax.ShapeDtypeStruct((E_HALF, 128), jnp.float32),
          jax.ShapeDtypeStruct((E_HALF,), jnp.float32),
          jax.ShapeDtypeStruct((E_HALF,), jnp.float32),
          jax.ShapeDtypeStruct((E_HALF,), jnp.float32),
      ],
      compiler_params=pltpu.CompilerParams(
          dimension_semantics=("arbitrary",)),
  )(g_arr, et_row, w2pi, w1s, w1c, w1ab, b1, w2, b2, ab, inb, etab, dw, db)


HALF_N = N_NODES // 2
ACC_ROWS = 6144
W_ROWS = 312
PER_T2 = E_HALF // NS


def _make_scatter_body(off_e, has_prev):
  def _scatter_body(zeros_hbm, msgl, msgr, eqxa, eqya, eqza, colp,
                    pfl, pfr, peq,
                    featl, featr, eqout, speq,
                    idxv, idxw0, idxw1, rows,
                    eqb0, eqb1, eqb2, eqa0, eqa1, eqa2,
                    zsrc, eqred, acc, seml, sems):
    cid = lax.axis_index("c")
    t = lax.axis_index("s")

    pltpu.sync_copy(zeros_hbm, zsrc)

    def zeq(i, carry):
      z = jnp.zeros((16,), jnp.float32)
      eqa0[pl.ds(i * 16, 16)] = z
      eqa1[pl.ds(i * 16, 16)] = z
      eqa2[pl.ds(i * 16, 16)] = z
      return carry
    lax.fori_loop(0, N_NODES // 16, zeq, 0)

    start = t * PER_T2
    n_ch = jnp.maximum(
        jnp.minimum(N_EDGES - off_e - start, PER_T2), 0) // 128

    idxws = (idxw0, idxw1)

    def run(msgx, h, with_eq):
      def mapidx(idxw, voff):
        for k in range(8):
          sl16 = pl.ds(voff + k * 16, 16)
          c16 = idxv[sl16]
          local = c16 - h * HALF_N
          valid = jnp.logical_and(local >= 0, local < HALF_N)
          idxw[pl.ds(k * 16, 16)] = jnp.where(
              valid, local, HALF_N + jnp.bitwise_and(c16, 1023))
          if with_eq:
            plsc.addupdate_scatter(eqa0, [c16], eqb0[sl16])
            plsc.addupdate_scatter(eqa1, [c16], eqb1[sl16])
            plsc.addupdate_scatter(eqa2, [c16], eqb2[sl16])

      def batch(b, carry):
        base = start + b * 256
        dl = [pltpu.async_copy(colp.at[pl.ds(off_e + base, 256)], idxv,
                               seml)]
        for q in range(2):
          dl.append(pltpu.async_copy(
              msgx.at[pl.ds(base + q * 128, 128), :],
              rows.at[pl.ds(q * 128, 128)], seml))
        if with_eq:
          dl.append(pltpu.async_copy(eqxa.at[pl.ds(base, 256)], eqb0, seml))
          dl.append(pltpu.async_copy(eqya.at[pl.ds(base, 256)], eqb1, seml))
          dl.append(pltpu.async_copy(eqza.at[pl.ds(base, 256)], eqb2, seml))
        for d in dl:
          d.wait()
        ds_ = []
        for q in range(2):
          mapidx(idxws[q], q * 128)
          ds_.append(pltpu.async_copy(rows.at[pl.ds(q * 128, 128)],
                                      acc.at[idxws[q]], sems, add=True))
        for d in ds_:
          d.wait()
        return carry
      lax.fori_loop(0, n_ch // 2, batch, 0)

      def chunk(j, carry):
        base = start + j * 128
        pltpu.sync_copy(colp.at[pl.ds(off_e + base, 128)],
                        idxv.at[pl.ds(0, 128)])
        pltpu.sync_copy(msgx.at[pl.ds(base, 128), :],
                        rows.at[pl.ds(0, 128)])
        if with_eq:
          pltpu.sync_copy(eqxa.at[pl.ds(base, 128)], eqb0.at[pl.ds(0, 128)])
          pltpu.sync_copy(eqya.at[pl.ds(base, 128)], eqb1.at[pl.ds(0, 128)])
          pltpu.sync_copy(eqza.at[pl.ds(base, 128)], eqb2.at[pl.ds(0, 128)])
        mapidx(idxw0, 0)
        pltpu.sync_copy(rows.at[pl.ds(0, 128)], acc.at[idxw0], add=True)
        return carry
      lax.fori_loop((n_ch // 2) * 2, n_ch, chunk, 0)

    for h in range(2):
      def init_zero(rlo, n128s, rem=0):
        for j in range(n128s):
          pltpu.sync_copy(zsrc, acc.at[pl.ds(rlo + j * 128, 128)])
        if rem:
          pltpu.sync_copy(zsrc.at[pl.ds(0, rem)],
                          acc.at[pl.ds(rlo + n128s * 128, rem)])

      if not has_prev:
        init_zero(t * 384, 3)
      else:
        def init_prev(px):
          @pl.when(t < 13)
          def _():
            for j in range(3):
              pltpu.sync_copy(
                  px.at[pl.ds(h * HALF_N + t * 384 + j * 128, 128)],
                  acc.at[pl.ds(t * 384 + j * 128, 128)])
          @pl.when(t == 13)
          def _():
            pltpu.sync_copy(px.at[pl.ds(h * HALF_N + 4992, 8)],
                            acc.at[pl.ds(4992, 8)])
            init_zero(5000, 2, 120)
          @pl.when(t > 13)
          def _():
            init_zero(t * 384, 3)
        @pl.when(cid == 0)
        def _():
          init_prev(pfl)
        @pl.when(cid == 1)
        def _():
          init_prev(pfr)
      plsc.subcore_barrier()

      @pl.when(cid == 0)
      def _():
        run(msgl, h, with_eq=(h == 0))
        if h == 0:
          pltpu.sync_copy(eqa0, speq.at[pl.ds(t * 3 * N_NODES, N_NODES)])
          pltpu.sync_copy(eqa1,
                          speq.at[pl.ds(t * 3 * N_NODES + N_NODES,
                                        N_NODES)])
          pltpu.sync_copy(eqa2,
                          speq.at[pl.ds(t * 3 * N_NODES + 2 * N_NODES,
                                        N_NODES)])

      @pl.when(cid == 1)
      def _():
        run(msgr, h, with_eq=False)

      plsc.subcore_barrier()

      def writeout(featx, rstart, nrows):
        pltpu.sync_copy(acc.at[pl.ds(rstart, nrows)],
                        featx.at[pl.ds(h * HALF_N + rstart, nrows)])

      @pl.when(jnp.logical_and(cid == 0, t < NS - 1))
      def _():
        writeout(featl, t * W_ROWS, W_ROWS)
      @pl.when(jnp.logical_and(cid == 0, t == NS - 1))
      def _():
        writeout(featl, (NS - 1) * W_ROWS, HALF_N - (NS - 1) * W_ROWS)
      @pl.when(jnp.logical_and(cid == 1, t < NS - 1))
      def _():
        writeout(featr, t * W_ROWS, W_ROWS)
      @pl.when(jnp.logical_and(cid == 1, t == NS - 1))
      def _():
        writeout(featr, (NS - 1) * W_ROWS, HALF_N - (NS - 1) * W_ROWS)
      plsc.subcore_barrier()

    def eq_reduce(nbase, cnt):
      for r, eqa in ((0, eqa0), (1, eqa1), (2, eqa2)):
        def addv(i, carry):
          sl16 = pl.ds(i * 16, 16)
          eqa[sl16] = eqa[sl16] + eqred[sl16]
          return carry

        if has_prev:
          pltpu.sync_copy(peq.at[pl.ds(r * N_NODES + nbase, cnt)],
                          eqa.at[pl.ds(0, cnt)])
          srcs = range(NS)
        else:
          pltpu.sync_copy(speq.at[pl.ds(r * N_NODES + nbase, cnt)],
                          eqa.at[pl.ds(0, cnt)])
          srcs = range(1, NS)
        for src in srcs:
          pltpu.sync_copy(
              speq.at[pl.ds((src * 3 + r) * N_NODES + nbase, cnt)],
              eqred.at[pl.ds(0, cnt)])
          lax.fori_loop(0, cnt // 16, addv, 0)
        pltpu.sync_copy(eqa.at[pl.ds(0, cnt)],
                        eqout.at[pl.ds(r * N_NODES + nbase, cnt)])

    @pl.when(jnp.logical_and(cid == 0, t < NS - 1))
    def _():
      eq_reduce(t * 640, 640)

    @pl.when(jnp.logical_and(cid == 0, t == NS - 1))
    def _():
      eq_reduce((NS - 1) * 640, 400)

  return _scatter_body


def _sc_scatter(zeros, msgl, msgr, eqxa, eqya, eqza, colp, pfl, pfr, peq,
                off_e, has_prev):
  k = functools.partial(
      pl.kernel,
      out_type=(
          jax.ShapeDtypeStruct((N_NODES, 128), jnp.float32),
          jax.ShapeDtypeStruct((N_NODES, 128), jnp.float32),
          jax.ShapeDtypeStruct((3 * N_NODES,), jnp.float32),
          jax.ShapeDtypeStruct((NS * 3 * N_NODES,), jnp.float32),
      ),
      mesh=_mesh(),
      scratch_types=[
          pltpu.VMEM((256,), jnp.int32),
          pltpu.VMEM((128,), jnp.int32),
          pltpu.VMEM((128,), jnp.int32),
          pltpu.VMEM((256, 128), jnp.float32),
          pltpu.VMEM((256,), jnp.float32),
          pltpu.VMEM((256,), jnp.float32),
          pltpu.VMEM((256,), jnp.float32),
          pltpu.VMEM((N_NODES,), jnp.float32),
          pltpu.VMEM((N_NODES,), jnp.float32),
          pltpu.VMEM((N_NODES,), jnp.float32),
          pltpu.VMEM((128, 128), jnp.float32),
          pltpu.VMEM((640,), jnp.float32),
          pltpu.VMEM_SHARED((ACC_ROWS, 128), jnp.float32),
          pltpu.SemaphoreType.DMA,
          pltpu.SemaphoreType.DMA,
      ],
      compiler_params=pltpu.CompilerParams(needs_layout_passes=False),
  )(_make_scatter_body(off_e, has_prev))
  return k(zeros, msgl, msgr, eqxa, eqya, eqza, colp, pfl, pfr, peq)


def kernel(pos, edge_index, node_type, edge_type, W_fourier, node_table,
           edge_table, in_W, in_b, dist_W, dist_b, proj_W1, proj_b1, proj_W2,
           proj_b2):
  f32 = jnp.float32
  pad_e = E_PAD - N_EDGES
  row = edge_index[0]
  col = edge_index[1]
  rowp = jnp.concatenate([row, jnp.zeros((pad_e,), row.dtype)])
  colp = jnp.concatenate([col, jnp.zeros((pad_e,), col.dtype)])
  posx, posy, posz = pos[:, 0], pos[:, 1], pos[:, 2]
  ntf = node_type.astype(f32)
  et_row = jnp.pad(edge_type.astype(f32), (0, pad_e)).reshape(1, E_PAD)

  ntp = jnp.concatenate([node_table, jnp.zeros((28, HID), f32)], axis=0)
  etab = jnp.concatenate([edge_table, jnp.zeros((28, HID), f32)], axis=0)
  w_cat = jnp.concatenate([in_W[:HID], in_W[HID:]], axis=1)
  w2pi = (W_fourier * (2.0 * jnp.pi)).reshape(1, HID)
  w1s = proj_W1[:HID].astype(jnp.bfloat16)
  w1c = proj_W1[HID:2 * HID].astype(jnp.bfloat16)
  w2b = proj_W2.astype(jnp.bfloat16)
  w1ab = proj_W1[2 * HID:]
  b1 = proj_b1.reshape(1, HID)
  b2 = proj_b2.reshape(1, HID)
  inb = in_b.reshape(1, HID)
  db = dist_b.reshape(1, HID)

  ab = _tc_prep(ntp, w_cat)
  g_arr = _sc_gather(posx, posy, posz, ntf, rowp, colp)
  zeros = jnp.zeros((128, 128), f32)
  m1 = _tc_main(g_arr, et_row, w2pi, w1s, w1c, w1ab, b1, w2b, b2, ab,
                inb, etab, dist_W, db, 0)
  m2 = _tc_main(g_arr, et_row, w2pi, w1s, w1c, w1ab, b1, w2b, b2, ab,
                inb, etab, dist_W, db, E_HALF // EB)
  fl1, fr1, eq1, sp1 = _sc_scatter(zeros, *m1, colp, zeros, zeros, zeros,
                                   0, False)
  featl, featr, eqout, _ = _sc_scatter(zeros, *m2, colp, fl1, fr1, eq1,
                                       E_HALF, True)
  return jnp.concatenate([featl, featr, eqout.reshape(3, N_NODES).T], axis=1)
